# baseline (speedup 1.0000x reference)
import functools

import jax
import jax.numpy as jnp
from jax import lax
from jax.experimental import pallas as pl
from jax.experimental.pallas import tpu as pltpu
from jax.experimental.pallas import tpu_sc as plsc

EPS = 1e-6
SCALE = 30.0
NEG_SLOPE = 0.01

NC = 2
NS = 16
NW = NC * NS
C = 80
NB = 8
L = 16


def _lrelu(x):
    return jnp.where(x >= 0, x, NEG_SLOPE * x)


def _layernorm(x, g, b, h):
    mu = jnp.mean(x, axis=1, keepdims=True)
    d = x - mu
    var = jnp.sum(d * d, axis=1, keepdims=True) / (h - 1)
    sigma = jnp.sqrt(var + EPS)
    return g * d / (sigma + EPS) + b


def _gather(h_v, src, e_total):
    n, h = h_v.shape
    per_w = e_total // NW
    c = C if per_w % C == 0 else C // 2
    chunks = per_w // c
    mesh = plsc.VectorSubcoreMesh(core_axis_name="c", subcore_axis_name="s")

    @functools.partial(
        pl.kernel,
        out_type=jax.ShapeDtypeStruct((e_total, h), jnp.float32),
        mesh=mesh,
        scratch_types=[
            pltpu.VMEM((c,), jnp.int32),
            pltpu.VMEM((c, h), jnp.float32),
            pltpu.SemaphoreType.DMA,
        ],
    )
    def k(hv_hbm, src_hbm, out_hbm, idx_v, rows_v, sem):
        wid = lax.axis_index("s") * NC + lax.axis_index("c")
        base = wid * per_w

        def body(j, carry):
            e0 = base + j * c
            pltpu.sync_copy(src_hbm.at[pl.ds(e0, c)], idx_v)
            pltpu.async_copy(hv_hbm.at[idx_v], rows_v, sem).wait()
            pltpu.sync_copy(rows_v, out_hbm.at[pl.ds(e0, c)])
            return carry

        lax.fori_loop(0, chunks, body, 0)

    return k(h_v, src)


def _edge_body(hvg_ref, he_ref, w1a_ref, w1b_ref, b1_ref, w2_ref, b2_ref,
               w3_ref, b3_ref, aa_ref, ab_ref, out_ref, att_ref):
    x = hvg_ref[...]
    e = he_ref[...]
    xb = x.astype(jnp.bfloat16)
    eb = e.astype(jnp.bfloat16)
    pre = (jnp.dot(xb, w1a_ref[...].astype(jnp.bfloat16),
                   preferred_element_type=jnp.float32)
           + jnp.dot(eb, w1b_ref[...].astype(jnp.bfloat16),
                     preferred_element_type=jnp.float32)
           + b1_ref[...])
    h1 = _lrelu(pre)
    h2 = _lrelu(jnp.dot(h1.astype(jnp.bfloat16),
                        w2_ref[...].astype(jnp.bfloat16),
                        preferred_element_type=jnp.float32)
                + b2_ref[...])
    hm = jnp.dot(h2.astype(jnp.bfloat16), w3_ref[...].astype(jnp.bfloat16),
                 preferred_element_type=jnp.float32) + b3_ref[...]
    logit = (jnp.sum(x * aa_ref[...], axis=1, keepdims=True)
             + jnp.sum(e * ab_ref[...], axis=1, keepdims=True))
    att = jnp.exp(jax.nn.sigmoid(_lrelu(logit)))
    out_ref[...] = hm * att
    att_ref[...] = att


def _edge_mlp(hvg, h_e, w1a, w1b, b1, w2, b2, w3, b3, aa, ab):
    e_total, h = hvg.shape
    nin = h_e.shape[1]
    be = 2000
    grid = e_total // be
    return pl.pallas_call(
        _edge_body,
        grid=(grid,),
        in_specs=[
            pl.BlockSpec((be, h), lambda i: (i, 0)),
            pl.BlockSpec((be, nin), lambda i: (i, 0)),
            pl.BlockSpec((h, h), lambda i: (0, 0)),
            pl.BlockSpec((nin, h), lambda i: (0, 0)),
            pl.BlockSpec((1, h), lambda i: (0, 0)),
            pl.BlockSpec((h, h), lambda i: (0, 0)),
            pl.BlockSpec((1, h), lambda i: (0, 0)),
            pl.BlockSpec((h, h), lambda i: (0, 0)),
            pl.BlockSpec((1, h), lambda i: (0, 0)),
            pl.BlockSpec((1, h), lambda i: (0, 0)),
            pl.BlockSpec((1, nin), lambda i: (0, 0)),
        ],
        out_specs=[
            pl.BlockSpec((be, h), lambda i: (i, 0)),
            pl.BlockSpec((be, 1), lambda i: (i, 0)),
        ],
        out_shape=[
            jax.ShapeDtypeStruct((e_total, h), jnp.float32),
            jax.ShapeDtypeStruct((e_total, 1), jnp.float32),
        ],
    )(hvg, h_e, w1a, w1b, b1, w2, b2, w3, b3, aa, ab)


def _scatter(payload, src, zeros, n_pad):
    e_total = src.shape[0]
    h = payload.shape[1]
    per_sc = e_total // NC
    per_tile = per_sc // NS
    c = C if per_tile % C == 0 else C // 2
    chunks = per_tile // c
    n_per_tile = n_pad // NS
    mesh = plsc.VectorSubcoreMesh(core_axis_name="c", subcore_axis_name="s")

    @functools.partial(
        pl.kernel,
        out_type=[
            jax.ShapeDtypeStruct((n_pad, h), jnp.float32),
            jax.ShapeDtypeStruct((n_pad, h), jnp.float32),
        ],
        mesh=mesh,
        scratch_types=[
            pltpu.VMEM((c,), jnp.int32),
            pltpu.VMEM((c, h), jnp.float32),
            pltpu.VMEM_SHARED((n_pad, h), jnp.float32),
        ],
    )
    def k(pay_hbm, src_hbm, zero_hbm, out_a, out_b, idx_v, pay_v, acc):
        cid = lax.axis_index("c")
        sid = lax.axis_index("s")
        sl = pl.ds(sid * n_per_tile, n_per_tile)
        pltpu.sync_copy(zero_hbm.at[sl], acc.at[sl])
        plsc.subcore_barrier()
        base = cid * per_sc + sid * per_tile

        def body(j, carry):
            e0 = base + j * c
            pltpu.sync_copy(src_hbm.at[pl.ds(e0, c)], idx_v)
            pltpu.sync_copy(pay_hbm.at[pl.ds(e0, c)], pay_v)
            pltpu.sync_copy(pay_v, acc.at[idx_v], add=True)
            return carry

        lax.fori_loop(0, chunks, body, 0)
        plsc.subcore_barrier()

        @pl.when(cid == 0)
        def _():
            pltpu.sync_copy(acc.at[sl], out_a.at[sl])

        @pl.when(cid == 1)
        def _():
            pltpu.sync_copy(acc.at[sl], out_b.at[sl])

    return k(payload, src, zeros)


def _att_hist(att, src, n_pad):
    e_total = src.shape[0]
    per_tile = e_total // NW
    chunks = per_tile // C
    mesh = plsc.VectorSubcoreMesh(core_axis_name="c", subcore_axis_name="s")

    @functools.partial(
        pl.kernel,
        out_type=jax.ShapeDtypeStruct((NW, NB, n_pad), jnp.float32),
        mesh=mesh,
        scratch_types=[
            pltpu.VMEM((C,), jnp.int32),
            pltpu.VMEM((C,), jnp.float32),
            pltpu.VMEM((NB * n_pad,), jnp.float32),
        ],
        compiler_params=pltpu.CompilerParams(needs_layout_passes=False),
    )
    def k(att_hbm, src_hbm, t_out, idx_v, att_v, tacc):
        cid = lax.axis_index("c")
        sid = lax.axis_index("s")
        wid = sid * NC + cid

        def zbody(i, carry):
            tacc[pl.ds(i * L, L)] = jnp.zeros((L,), jnp.float32)
            return carry

        lax.fori_loop(0, NB * n_pad // L, zbody, 0)

        base = wid * per_tile
        lane = lax.iota(jnp.int32, L)
        bank_off = (lane % NB) * n_pad
        m_lo = lane < NB
        m_hi = lane >= NB

        def body(j, carry):
            e0 = base + j * C
            pltpu.sync_copy(src_hbm.at[pl.ds(e0, C)], idx_v)
            pltpu.sync_copy(att_hbm.at[pl.ds(e0, C)], att_v)
            for g in range(C // L):
                flat = idx_v[pl.ds(g * L, L)] + bank_off
                att16 = att_v[pl.ds(g * L, L)]
                plsc.addupdate_scatter(tacc, [flat], att16, mask=m_lo)
                plsc.addupdate_scatter(tacc, [flat], att16, mask=m_hi)
            return carry

        lax.fori_loop(0, chunks, body, 0)

        for b in range(NB):
            pltpu.sync_copy(tacc.at[pl.ds(b * n_pad, n_pad)], t_out.at[wid, b])

    return k(att, src)


def _final_body(hv_ref, acca_ref, accb_ref, accc_ref, accd_ref, t_ref,
                g0_ref, beta0_ref, g1_ref, beta1_ref, win_ref, bin_ref,
                wout_ref, bout_ref, out_ref):
    h = hv_ref.shape[1]
    s = ((acca_ref[...] + accb_ref[...]) + (accc_ref[...] + accd_ref[...]))
    t = jnp.sum(t_ref[...], axis=1, keepdims=True)
    dh = jnp.where(t > 0, s / jnp.where(t > 0, t, 1.0), 0.0) / SCALE
    x = _layernorm(hv_ref[...] + dh, g0_ref[...], beta0_ref[...], h)
    y = jnp.maximum(
        jnp.dot(x, win_ref[...], preferred_element_type=jnp.float32)
        + bin_ref[...], 0.0)
    y = jnp.dot(y, wout_ref[...], preferred_element_type=jnp.float32) + bout_ref[...]
    out_ref[...] = _layernorm(x + y, g1_ref[...], beta1_ref[...], h)


def _finalize(h_v, accs, t_part, g0, beta0, g1, beta1, win, bin_,
              wout, bout):
    n, h = h_v.shape
    bn = 1000
    grid = n // bn
    h4 = win.shape[1]
    kp = t_part.shape[1]
    return pl.pallas_call(
        _final_body,
        grid=(grid,),
        in_specs=[
            pl.BlockSpec((bn, h), lambda i: (i, 0)),
            pl.BlockSpec((bn, h), lambda i: (i, 0)),
            pl.BlockSpec((bn, h), lambda i: (i, 0)),
            pl.BlockSpec((bn, h), lambda i: (i, 0)),
            pl.BlockSpec((bn, h), lambda i: (i, 0)),
            pl.BlockSpec((bn, kp), lambda i: (i, 0)),
            pl.BlockSpec((1, h), lambda i: (0, 0)),
            pl.BlockSpec((1, h), lambda i: (0, 0)),
            pl.BlockSpec((1, h), lambda i: (0, 0)),
            pl.BlockSpec((1, h), lambda i: (0, 0)),
            pl.BlockSpec((h, h4), lambda i: (0, 0)),
            pl.BlockSpec((1, h4), lambda i: (0, 0)),
            pl.BlockSpec((h4, h), lambda i: (0, 0)),
            pl.BlockSpec((1, h), lambda i: (0, 0)),
        ],
        out_specs=pl.BlockSpec((bn, h), lambda i: (i, 0)),
        out_shape=jax.ShapeDtypeStruct((n, h), jnp.float32),
    )(h_v, *accs, t_part, g0, beta0, g1, beta1, win, bin_, wout, bout)


def kernel(h_V, h_E, edge_idx, W1, b1, W2, b2, W3, b3, A, g0, beta0, g1,
           beta1, Win, bin, Wout, bout):
    n, h = h_V.shape
    e_total, nin = h_E.shape
    src = edge_idx[0]
    n_pad = ((n + NS * 8 - 1) // (NS * 8)) * NS * 8
    eh = e_total // 2
    zeros = jnp.zeros((n_pad, h), jnp.float32)
    ws = (W1[:h], W1[h:], b1.reshape(1, h), W2, b2.reshape(1, h),
          W3, b3.reshape(1, h), A[:h].reshape(1, h), A[h:].reshape(1, nin))

    accs = []
    atts = []
    for lo in (0, eh):
        src_h = lax.dynamic_slice_in_dim(src, lo, eh)
        hvg = _gather(h_V, src_h, eh)
        weighted, att = _edge_mlp(hvg, lax.dynamic_slice_in_dim(h_E, lo, eh),
                                  *ws)
        acc_a, acc_b = _scatter(weighted, src_h, zeros, n_pad)
        accs += [acc_a[:n], acc_b[:n]]
        atts.append(att.reshape(eh))
    t_out = _att_hist(jnp.concatenate(atts), src, n_pad)
    t_part = t_out.reshape(NW * NB, n_pad).T
    return _finalize(
        h_V, accs, t_part[:n],
        g0.reshape(1, h), beta0.reshape(1, h),
        g1.reshape(1, h), beta1.reshape(1, h),
        Win, bin.reshape(1, -1), Wout, bout.reshape(1, h),
    )


# speedup vs baseline: 1.4664x; 1.4664x over previous
import functools

import jax
import jax.numpy as jnp
from jax import lax
from jax.experimental import pallas as pl
from jax.experimental.pallas import tpu as pltpu
from jax.experimental.pallas import tpu_sc as plsc

EPS = 1e-6
SCALE = 30.0
NEG_SLOPE = 0.01

NC = 2
NS = 16
NW = NC * NS
C = 80
NB = 8
L = 16
KS = 4


def _lrelu(x):
    return jnp.where(x >= 0, x, NEG_SLOPE * x)


def _layernorm(x, g, b, h):
    mu = jnp.mean(x, axis=1, keepdims=True)
    d = x - mu
    var = jnp.sum(d * d, axis=1, keepdims=True) / (h - 1)
    sigma = jnp.sqrt(var + EPS)
    return g * d / (sigma + EPS) + b


KG = 5


def _gather(h_v, src, e_total):
    n, h = h_v.shape
    per_w = e_total // NW
    chunks = per_w // C
    iters = chunks // KG
    mesh = plsc.VectorSubcoreMesh(core_axis_name="c", subcore_axis_name="s")

    @functools.partial(
        pl.kernel,
        out_type=jax.ShapeDtypeStruct((e_total, h), jnp.float32),
        mesh=mesh,
        scratch_types=(
            [pltpu.VMEM((C,), jnp.int32) for _ in range(KG)]
            + [pltpu.VMEM((C, h), jnp.float32) for _ in range(KG)]
            + [pltpu.SemaphoreType.DMA]
            + [pltpu.SemaphoreType.DMA for _ in range(2 * KG)]
        ),
    )
    def k(hv_hbm, src_hbm, out_hbm, *scratch):
        idx_v = scratch[:KG]
        rows_v = scratch[KG:2 * KG]
        sem_i = scratch[2 * KG]
        sem_g = scratch[2 * KG + 1:2 * KG + 1 + KG]
        sem_w = scratch[2 * KG + 1 + KG:]
        wid = lax.axis_index("s") * NC + lax.axis_index("c")
        base = wid * per_w

        def body(t, carry):
            e0 = base + t * (KG * C)
            di = []
            for kk in range(KG):
                di.append(pltpu.async_copy(
                    src_hbm.at[pl.ds(e0 + kk * C, C)], idx_v[kk], sem_i))
            for kk in range(KG):
                di[kk].wait()
            dg = []
            for kk in range(KG):
                dg.append(pltpu.async_copy(
                    hv_hbm.at[idx_v[kk]], rows_v[kk], sem_g[kk]))
            dw = []
            for kk in range(KG):
                dg[kk].wait()
                dw.append(pltpu.async_copy(
                    rows_v[kk], out_hbm.at[pl.ds(e0 + kk * C, C)], sem_w[kk]))
            for kk in range(KG):
                dw[kk].wait()
            return carry

        lax.fori_loop(0, iters, body, 0)

    return k(h_v, src)


def _edge_body(hvg_ref, he_ref, w1a_ref, w1b_ref, b1_ref, w2_ref, b2_ref,
               w3_ref, b3_ref, aa_ref, ab_ref, out_ref, att_ref):
    x = hvg_ref[...]
    e = he_ref[...]
    pre = (jnp.dot(x, w1a_ref[...], preferred_element_type=jnp.float32)
           + jnp.dot(e, w1b_ref[...], preferred_element_type=jnp.float32)
           + b1_ref[...])
    h1 = _lrelu(pre)
    h2 = _lrelu(jnp.dot(h1, w2_ref[...], preferred_element_type=jnp.float32)
                + b2_ref[...])
    hm = jnp.dot(h2, w3_ref[...], preferred_element_type=jnp.float32) + b3_ref[...]
    logit = (jnp.sum(x * aa_ref[...], axis=1, keepdims=True)
             + jnp.sum(e * ab_ref[...], axis=1, keepdims=True))
    att = jnp.exp(jax.nn.sigmoid(_lrelu(logit)))
    out_ref[...] = hm * att
    att_ref[...] = att


def _edge_mlp(hvg, h_e, w1a, w1b, b1, w2, b2, w3, b3, aa, ab):
    e_total, h = hvg.shape
    nin = h_e.shape[1]
    be = 2000
    grid = e_total // be
    return pl.pallas_call(
        _edge_body,
        grid=(grid,),
        in_specs=[
            pl.BlockSpec((be, h), lambda i: (i, 0)),
            pl.BlockSpec((be, nin), lambda i: (i, 0)),
            pl.BlockSpec((h, h), lambda i: (0, 0)),
            pl.BlockSpec((nin, h), lambda i: (0, 0)),
            pl.BlockSpec((1, h), lambda i: (0, 0)),
            pl.BlockSpec((h, h), lambda i: (0, 0)),
            pl.BlockSpec((1, h), lambda i: (0, 0)),
            pl.BlockSpec((h, h), lambda i: (0, 0)),
            pl.BlockSpec((1, h), lambda i: (0, 0)),
            pl.BlockSpec((1, h), lambda i: (0, 0)),
            pl.BlockSpec((1, nin), lambda i: (0, 0)),
        ],
        out_specs=[
            pl.BlockSpec((be, h), lambda i: (i, 0)),
            pl.BlockSpec((be, 1), lambda i: (i, 0)),
        ],
        out_shape=[
            jax.ShapeDtypeStruct((e_total, h), jnp.float32),
            jax.ShapeDtypeStruct((e_total, 1), jnp.float32),
        ],
    )(hvg, h_e, w1a, w1b, b1, w2, b2, w3, b3, aa, ab)


def _scatter(payload, src, zeros, n_pad):
    e_total = src.shape[0]
    h = payload.shape[1]
    per_sc = e_total // NC
    per_tile = per_sc // NS
    chunks = per_tile // C
    iters = chunks // KS
    tail = chunks - iters * KS
    n_per_tile = n_pad // NS
    mesh = plsc.VectorSubcoreMesh(core_axis_name="c", subcore_axis_name="s")

    @functools.partial(
        pl.kernel,
        out_type=[
            jax.ShapeDtypeStruct((n_pad, h), jnp.float32),
            jax.ShapeDtypeStruct((n_pad, h), jnp.float32),
        ],
        mesh=mesh,
        scratch_types=(
            [pltpu.VMEM((C,), jnp.int32) for _ in range(KS)]
            + [pltpu.VMEM((C, h), jnp.float32) for _ in range(KS)]
            + [pltpu.VMEM_SHARED((n_pad, h), jnp.float32)]
            + [pltpu.SemaphoreType.DMA, pltpu.SemaphoreType.DMA]
            + [pltpu.SemaphoreType.DMA for _ in range(KS)]
        ),
    )
    def k(pay_hbm, src_hbm, zero_hbm, out_a, out_b, *scratch):
        idx_v = scratch[:KS]
        pay_v = scratch[KS:2 * KS]
        acc = scratch[2 * KS]
        sem_i = scratch[2 * KS + 1]
        sem_p = scratch[2 * KS + 2]
        sem_a = scratch[2 * KS + 3:]
        cid = lax.axis_index("c")
        sid = lax.axis_index("s")
        sl = pl.ds(sid * n_per_tile, n_per_tile)
        pltpu.sync_copy(zero_hbm.at[sl], acc.at[sl])
        plsc.subcore_barrier()
        base = cid * per_sc + sid * per_tile

        def body(t, carry):
            e0 = base + t * (KS * C)
            d = []
            for kk in range(KS):
                d.append(pltpu.async_copy(
                    src_hbm.at[pl.ds(e0 + kk * C, C)], idx_v[kk], sem_i))
                d.append(pltpu.async_copy(
                    pay_hbm.at[pl.ds(e0 + kk * C, C)], pay_v[kk], sem_p))
            for dd in d:
                dd.wait()
            da = []
            for kk in range(KS):
                da.append(pltpu.async_copy(
                    pay_v[kk], acc.at[idx_v[kk]], sem_a[kk], add=True))
            for dd in da:
                dd.wait()
            return carry

        lax.fori_loop(0, iters, body, 0)
        for j in range(tail):
            e0 = base + (iters * KS + j) * C
            pltpu.sync_copy(src_hbm.at[pl.ds(e0, C)], idx_v[0])
            pltpu.sync_copy(pay_hbm.at[pl.ds(e0, C)], pay_v[0])
            pltpu.sync_copy(pay_v[0], acc.at[idx_v[0]], add=True)
        plsc.subcore_barrier()

        @pl.when(cid == 0)
        def _():
            pltpu.sync_copy(acc.at[sl], out_a.at[sl])

        @pl.when(cid == 1)
        def _():
            pltpu.sync_copy(acc.at[sl], out_b.at[sl])

    return k(payload, src, zeros)


def _att_hist(att, src, n_pad):
    e_total = src.shape[0]
    per_tile = e_total // NW
    chunks = per_tile // C
    mesh = plsc.VectorSubcoreMesh(core_axis_name="c", subcore_axis_name="s")

    @functools.partial(
        pl.kernel,
        out_type=jax.ShapeDtypeStruct((NW, NB, n_pad), jnp.float32),
        mesh=mesh,
        scratch_types=[
            pltpu.VMEM((C,), jnp.int32),
            pltpu.VMEM((C,), jnp.float32),
            pltpu.VMEM((NB * n_pad,), jnp.float32),
        ],
        compiler_params=pltpu.CompilerParams(needs_layout_passes=False),
    )
    def k(att_hbm, src_hbm, t_out, idx_v, att_v, tacc):
        cid = lax.axis_index("c")
        sid = lax.axis_index("s")
        wid = sid * NC + cid

        def zbody(i, carry):
            tacc[pl.ds(i * L, L)] = jnp.zeros((L,), jnp.float32)
            return carry

        lax.fori_loop(0, NB * n_pad // L, zbody, 0)

        base = wid * per_tile
        lane = lax.iota(jnp.int32, L)
        bank_off = (lane % NB) * n_pad
        m_lo = lane < NB
        m_hi = lane >= NB

        def body(j, carry):
            e0 = base + j * C
            pltpu.sync_copy(src_hbm.at[pl.ds(e0, C)], idx_v)
            pltpu.sync_copy(att_hbm.at[pl.ds(e0, C)], att_v)
            for g in range(C // L):
                flat = idx_v[pl.ds(g * L, L)] + bank_off
                att16 = att_v[pl.ds(g * L, L)]
                plsc.addupdate_scatter(tacc, [flat], att16, mask=m_lo)
                plsc.addupdate_scatter(tacc, [flat], att16, mask=m_hi)
            return carry

        lax.fori_loop(0, chunks, body, 0)

        for b in range(NB):
            pltpu.sync_copy(tacc.at[pl.ds(b * n_pad, n_pad)], t_out.at[wid, b])

    return k(att, src)


def _final_body(hv_ref, acca_ref, accb_ref, t_ref,
                g0_ref, beta0_ref, g1_ref, beta1_ref, win_ref, bin_ref,
                wout_ref, bout_ref, out_ref):
    h = hv_ref.shape[1]
    s = acca_ref[...] + accb_ref[...]
    t = jnp.sum(t_ref[...], axis=1, keepdims=True)
    dh = jnp.where(t > 0, s / jnp.where(t > 0, t, 1.0), 0.0) / SCALE
    x = _layernorm(hv_ref[...] + dh, g0_ref[...], beta0_ref[...], h)
    y = jnp.maximum(
        jnp.dot(x, win_ref[...], preferred_element_type=jnp.float32)
        + bin_ref[...], 0.0)
    y = jnp.dot(y, wout_ref[...], preferred_element_type=jnp.float32) + bout_ref[...]
    out_ref[...] = _layernorm(x + y, g1_ref[...], beta1_ref[...], h)


def _finalize(h_v, accs, t_part, g0, beta0, g1, beta1, win, bin_,
              wout, bout):
    n, h = h_v.shape
    bn = 1000
    grid = n // bn
    h4 = win.shape[1]
    kp = t_part.shape[1]
    return pl.pallas_call(
        _final_body,
        grid=(grid,),
        in_specs=[
            pl.BlockSpec((bn, h), lambda i: (i, 0)),
            pl.BlockSpec((bn, h), lambda i: (i, 0)),
            pl.BlockSpec((bn, h), lambda i: (i, 0)),
            pl.BlockSpec((bn, kp), lambda i: (i, 0)),
            pl.BlockSpec((1, h), lambda i: (0, 0)),
            pl.BlockSpec((1, h), lambda i: (0, 0)),
            pl.BlockSpec((1, h), lambda i: (0, 0)),
            pl.BlockSpec((1, h), lambda i: (0, 0)),
            pl.BlockSpec((h, h4), lambda i: (0, 0)),
            pl.BlockSpec((1, h4), lambda i: (0, 0)),
            pl.BlockSpec((h4, h), lambda i: (0, 0)),
            pl.BlockSpec((1, h), lambda i: (0, 0)),
        ],
        out_specs=pl.BlockSpec((bn, h), lambda i: (i, 0)),
        out_shape=jax.ShapeDtypeStruct((n, h), jnp.float32),
    )(h_v, *accs, t_part, g0, beta0, g1, beta1, win, bin_, wout, bout)


def kernel(h_V, h_E, edge_idx, W1, b1, W2, b2, W3, b3, A, g0, beta0, g1,
           beta1, Win, bin, Wout, bout):
    n, h = h_V.shape
    e_total, nin = h_E.shape
    src = edge_idx[0]
    n_pad = ((n + NS * 8 - 1) // (NS * 8)) * NS * 8
    zeros = jnp.zeros((n_pad, h), jnp.float32)
    ws = (W1[:h], W1[h:], b1.reshape(1, h), W2, b2.reshape(1, h),
          W3, b3.reshape(1, h), A[:h].reshape(1, h), A[h:].reshape(1, nin))

    hvg = _gather(h_V, src, e_total)
    weighted, att = _edge_mlp(hvg, h_E, *ws)
    acc_a, acc_b = _scatter(weighted, src, zeros, n_pad)
    t_out = _att_hist(att.reshape(e_total), src, n_pad)
    t_part = t_out.reshape(NW * NB, n_pad).T
    return _finalize(
        h_V, [acc_a[:n], acc_b[:n]], t_part[:n],
        g0.reshape(1, h), beta0.reshape(1, h),
        g1.reshape(1, h), beta1.reshape(1, h),
        Win, bin.reshape(1, -1), Wout, bout.reshape(1, h),
    )


# speedup vs baseline: 1.6288x; 1.1108x over previous
import functools

import jax
import jax.numpy as jnp
from jax import lax
from jax.experimental import pallas as pl
from jax.experimental.pallas import tpu as pltpu
from jax.experimental.pallas import tpu_sc as plsc

EPS = 1e-6
SCALE = 30.0
NEG_SLOPE = 0.01

NC = 2
NS = 16
NW = NC * NS
C = 80
NB = 8
L = 16
KS = 4


def _lrelu(x):
    return jnp.where(x >= 0, x, NEG_SLOPE * x)


def _layernorm(x, g, b, h):
    mu = jnp.mean(x, axis=1, keepdims=True)
    d = x - mu
    var = jnp.sum(d * d, axis=1, keepdims=True) / (h - 1)
    sigma = jnp.sqrt(var + EPS)
    return g * d / (sigma + EPS) + b


KG = 5


def _gather(h_v, src, e_total):
    n, h = h_v.shape
    per_w = e_total // NW
    chunks = per_w // C
    iters = chunks // KG
    mesh = plsc.VectorSubcoreMesh(core_axis_name="c", subcore_axis_name="s")

    @functools.partial(
        pl.kernel,
        out_type=jax.ShapeDtypeStruct((e_total, h), jnp.float32),
        mesh=mesh,
        scratch_types=(
            [pltpu.VMEM((C,), jnp.int32) for _ in range(KG)]
            + [pltpu.VMEM((C, h), jnp.float32) for _ in range(KG)]
            + [pltpu.SemaphoreType.DMA]
            + [pltpu.SemaphoreType.DMA for _ in range(2 * KG)]
        ),
    )
    def k(hv_hbm, src_hbm, out_hbm, *scratch):
        idx_v = scratch[:KG]
        rows_v = scratch[KG:2 * KG]
        sem_i = scratch[2 * KG]
        sem_g = scratch[2 * KG + 1:2 * KG + 1 + KG]
        sem_w = scratch[2 * KG + 1 + KG:]
        wid = lax.axis_index("s") * NC + lax.axis_index("c")
        base = wid * per_w

        def body(t, carry):
            e0 = base + t * (KG * C)
            di = []
            for kk in range(KG):
                di.append(pltpu.async_copy(
                    src_hbm.at[pl.ds(e0 + kk * C, C)], idx_v[kk], sem_i))
            for kk in range(KG):
                di[kk].wait()
            dg = []
            for kk in range(KG):
                dg.append(pltpu.async_copy(
                    hv_hbm.at[idx_v[kk]], rows_v[kk], sem_g[kk]))
            dw = []
            for kk in range(KG):
                dg[kk].wait()
                dw.append(pltpu.async_copy(
                    rows_v[kk], out_hbm.at[pl.ds(e0 + kk * C, C)], sem_w[kk]))
            for kk in range(KG):
                dw[kk].wait()
            return carry

        lax.fori_loop(0, iters, body, 0)

    return k(h_v, src)


def _edge_body(hvg_ref, he_ref, w1a_ref, w1b_ref, b1_ref, w2_ref, b2_ref,
               w3_ref, b3_ref, aa_ref, ab_ref, out_ref, att_ref):
    x = hvg_ref[...]
    e = he_ref[...]
    pre = (jnp.dot(x, w1a_ref[...], preferred_element_type=jnp.float32)
           + jnp.dot(e, w1b_ref[...], preferred_element_type=jnp.float32)
           + b1_ref[...])
    h1 = _lrelu(pre)
    h2 = _lrelu(jnp.dot(h1, w2_ref[...], preferred_element_type=jnp.float32)
                + b2_ref[...])
    hm = jnp.dot(h2, w3_ref[...], preferred_element_type=jnp.float32) + b3_ref[...]
    logit = (jnp.sum(x * aa_ref[...], axis=1, keepdims=True)
             + jnp.sum(e * ab_ref[...], axis=1, keepdims=True))
    att = jnp.exp(jax.nn.sigmoid(_lrelu(logit)))
    out_ref[...] = hm * att
    att_ref[...] = att


def _edge_mlp(hvg, h_e, w1a, w1b, b1, w2, b2, w3, b3, aa, ab):
    e_total, h = hvg.shape
    nin = h_e.shape[1]
    be = 2000
    grid = e_total // be
    return pl.pallas_call(
        _edge_body,
        grid=(grid,),
        in_specs=[
            pl.BlockSpec((be, h), lambda i: (i, 0)),
            pl.BlockSpec((be, nin), lambda i: (i, 0)),
            pl.BlockSpec((h, h), lambda i: (0, 0)),
            pl.BlockSpec((nin, h), lambda i: (0, 0)),
            pl.BlockSpec((1, h), lambda i: (0, 0)),
            pl.BlockSpec((h, h), lambda i: (0, 0)),
            pl.BlockSpec((1, h), lambda i: (0, 0)),
            pl.BlockSpec((h, h), lambda i: (0, 0)),
            pl.BlockSpec((1, h), lambda i: (0, 0)),
            pl.BlockSpec((1, h), lambda i: (0, 0)),
            pl.BlockSpec((1, nin), lambda i: (0, 0)),
        ],
        out_specs=[
            pl.BlockSpec((be, h), lambda i: (i, 0)),
            pl.BlockSpec((be, 1), lambda i: (i, 0)),
        ],
        out_shape=[
            jax.ShapeDtypeStruct((e_total, h), jnp.float32),
            jax.ShapeDtypeStruct((e_total, 1), jnp.float32),
        ],
    )(hvg, h_e, w1a, w1b, b1, w2, b2, w3, b3, aa, ab)


def _scatter(payload, src, zeros, n_pad):
    e_total = src.shape[0]
    h = payload.shape[1]
    per_sc = e_total // NC
    per_tile = per_sc // NS
    chunks = per_tile // C
    iters = chunks // KS
    tail = chunks - iters * KS
    n_per_tile = n_pad // NS
    mesh = plsc.VectorSubcoreMesh(core_axis_name="c", subcore_axis_name="s")

    @functools.partial(
        pl.kernel,
        out_type=[
            jax.ShapeDtypeStruct((n_pad, h), jnp.float32),
            jax.ShapeDtypeStruct((n_pad, h), jnp.float32),
        ],
        mesh=mesh,
        scratch_types=(
            [pltpu.VMEM((C,), jnp.int32) for _ in range(KS)]
            + [pltpu.VMEM((C, h), jnp.float32) for _ in range(KS)]
            + [pltpu.VMEM_SHARED((n_pad, h), jnp.float32)]
            + [pltpu.SemaphoreType.DMA, pltpu.SemaphoreType.DMA]
            + [pltpu.SemaphoreType.DMA for _ in range(KS)]
        ),
    )
    def k(pay_hbm, src_hbm, zero_hbm, out_a, out_b, *scratch):
        idx_v = scratch[:KS]
        pay_v = scratch[KS:2 * KS]
        acc = scratch[2 * KS]
        sem_i = scratch[2 * KS + 1]
        sem_p = scratch[2 * KS + 2]
        sem_a = scratch[2 * KS + 3:]
        cid = lax.axis_index("c")
        sid = lax.axis_index("s")
        sl = pl.ds(sid * n_per_tile, n_per_tile)
        pltpu.sync_copy(zero_hbm.at[sl], acc.at[sl])
        plsc.subcore_barrier()
        base = cid * per_sc + sid * per_tile

        def body(t, carry):
            e0 = base + t * (KS * C)
            d = []
            for kk in range(KS):
                d.append(pltpu.async_copy(
                    src_hbm.at[pl.ds(e0 + kk * C, C)], idx_v[kk], sem_i))
                d.append(pltpu.async_copy(
                    pay_hbm.at[pl.ds(e0 + kk * C, C)], pay_v[kk], sem_p))
            for dd in d:
                dd.wait()
            da = []
            for kk in range(KS):
                da.append(pltpu.async_copy(
                    pay_v[kk], acc.at[idx_v[kk]], sem_a[kk], add=True))
            for dd in da:
                dd.wait()
            return carry

        lax.fori_loop(0, iters, body, 0)
        for j in range(tail):
            e0 = base + (iters * KS + j) * C
            pltpu.sync_copy(src_hbm.at[pl.ds(e0, C)], idx_v[0])
            pltpu.sync_copy(pay_hbm.at[pl.ds(e0, C)], pay_v[0])
            pltpu.sync_copy(pay_v[0], acc.at[idx_v[0]], add=True)
        plsc.subcore_barrier()

        @pl.when(cid == 0)
        def _():
            pltpu.sync_copy(acc.at[sl], out_a.at[sl])

        @pl.when(cid == 1)
        def _():
            pltpu.sync_copy(acc.at[sl], out_b.at[sl])

    return k(payload, src, zeros)


def _att_hist(att, src, n_pad):
    e_total = src.shape[0]
    per_tile = e_total // NW
    chunks = per_tile // C
    iters = chunks // KS
    tail = chunks - iters * KS
    mesh = plsc.VectorSubcoreMesh(core_axis_name="c", subcore_axis_name="s")

    @functools.partial(
        pl.kernel,
        out_type=jax.ShapeDtypeStruct((NW, NB, n_pad), jnp.float32),
        mesh=mesh,
        scratch_types=(
            [pltpu.VMEM((C,), jnp.int32) for _ in range(KS)]
            + [pltpu.VMEM((C,), jnp.float32) for _ in range(KS)]
            + [pltpu.VMEM((NB * n_pad,), jnp.float32)]
            + [pltpu.SemaphoreType.DMA, pltpu.SemaphoreType.DMA]
        ),
        compiler_params=pltpu.CompilerParams(needs_layout_passes=False),
    )
    def k(att_hbm, src_hbm, t_out, *scratch):
        idx_v = scratch[:KS]
        att_v = scratch[KS:2 * KS]
        tacc = scratch[2 * KS]
        sem_i = scratch[2 * KS + 1]
        sem_a = scratch[2 * KS + 2]
        cid = lax.axis_index("c")
        sid = lax.axis_index("s")
        wid = sid * NC + cid

        def zbody(i, carry):
            tacc[pl.ds(i * L, L)] = jnp.zeros((L,), jnp.float32)
            return carry

        lax.fori_loop(0, NB * n_pad // L, zbody, 0)

        base = wid * per_tile
        lane = lax.iota(jnp.int32, L)
        bank_off = (lane % NB) * n_pad
        m_lo = lane < NB
        m_hi = lane >= NB

        def hist(kk):
            for g in range(C // L):
                flat = idx_v[kk][pl.ds(g * L, L)] + bank_off
                att16 = att_v[kk][pl.ds(g * L, L)]
                plsc.addupdate_scatter(tacc, [flat], att16, mask=m_lo)
                plsc.addupdate_scatter(tacc, [flat], att16, mask=m_hi)

        def body(t, carry):
            e0 = base + t * (KS * C)
            d = []
            for kk in range(KS):
                d.append(pltpu.async_copy(
                    src_hbm.at[pl.ds(e0 + kk * C, C)], idx_v[kk], sem_i))
                d.append(pltpu.async_copy(
                    att_hbm.at[pl.ds(e0 + kk * C, C)], att_v[kk], sem_a))
            for dd in d:
                dd.wait()
            for kk in range(KS):
                hist(kk)
            return carry

        lax.fori_loop(0, iters, body, 0)
        for j in range(tail):
            e0 = base + (iters * KS + j) * C
            pltpu.sync_copy(src_hbm.at[pl.ds(e0, C)], idx_v[0])
            pltpu.sync_copy(att_hbm.at[pl.ds(e0, C)], att_v[0])
            hist(0)

        for b in range(NB):
            pltpu.sync_copy(tacc.at[pl.ds(b * n_pad, n_pad)], t_out.at[wid, b])

    return k(att, src)


def _final_body(hv_ref, acca_ref, accb_ref, t_ref,
                g0_ref, beta0_ref, g1_ref, beta1_ref, win_ref, bin_ref,
                wout_ref, bout_ref, out_ref):
    h = hv_ref.shape[1]
    s = acca_ref[...] + accb_ref[...]
    t = jnp.sum(t_ref[...], axis=1, keepdims=True)
    dh = jnp.where(t > 0, s / jnp.where(t > 0, t, 1.0), 0.0) / SCALE
    x = _layernorm(hv_ref[...] + dh, g0_ref[...], beta0_ref[...], h)
    y = jnp.maximum(
        jnp.dot(x, win_ref[...], preferred_element_type=jnp.float32)
        + bin_ref[...], 0.0)
    y = jnp.dot(y, wout_ref[...], preferred_element_type=jnp.float32) + bout_ref[...]
    out_ref[...] = _layernorm(x + y, g1_ref[...], beta1_ref[...], h)


def _finalize(h_v, accs, t_part, g0, beta0, g1, beta1, win, bin_,
              wout, bout):
    n, h = h_v.shape
    bn = 1000
    grid = n // bn
    h4 = win.shape[1]
    kp = t_part.shape[1]
    return pl.pallas_call(
        _final_body,
        grid=(grid,),
        in_specs=[
            pl.BlockSpec((bn, h), lambda i: (i, 0)),
            pl.BlockSpec((bn, h), lambda i: (i, 0)),
            pl.BlockSpec((bn, h), lambda i: (i, 0)),
            pl.BlockSpec((bn, kp), lambda i: (i, 0)),
            pl.BlockSpec((1, h), lambda i: (0, 0)),
            pl.BlockSpec((1, h), lambda i: (0, 0)),
            pl.BlockSpec((1, h), lambda i: (0, 0)),
            pl.BlockSpec((1, h), lambda i: (0, 0)),
            pl.BlockSpec((h, h4), lambda i: (0, 0)),
            pl.BlockSpec((1, h4), lambda i: (0, 0)),
            pl.BlockSpec((h4, h), lambda i: (0, 0)),
            pl.BlockSpec((1, h), lambda i: (0, 0)),
        ],
        out_specs=pl.BlockSpec((bn, h), lambda i: (i, 0)),
        out_shape=jax.ShapeDtypeStruct((n, h), jnp.float32),
    )(h_v, *accs, t_part, g0, beta0, g1, beta1, win, bin_, wout, bout)


def kernel(h_V, h_E, edge_idx, W1, b1, W2, b2, W3, b3, A, g0, beta0, g1,
           beta1, Win, bin, Wout, bout):
    n, h = h_V.shape
    e_total, nin = h_E.shape
    src = edge_idx[0]
    n_pad = ((n + NS * 8 - 1) // (NS * 8)) * NS * 8
    zeros = jnp.zeros((n_pad, h), jnp.float32)
    ws = (W1[:h], W1[h:], b1.reshape(1, h), W2, b2.reshape(1, h),
          W3, b3.reshape(1, h), A[:h].reshape(1, h), A[h:].reshape(1, nin))

    hvg = _gather(h_V, src, e_total)
    weighted, att = _edge_mlp(hvg, h_E, *ws)
    acc_a, acc_b = _scatter(weighted, src, zeros, n_pad)
    t_out = _att_hist(att.reshape(e_total), src, n_pad)
    t_part = t_out.reshape(NW * NB, n_pad).T
    return _finalize(
        h_V, [acc_a[:n], acc_b[:n]], t_part[:n],
        g0.reshape(1, h), beta0.reshape(1, h),
        g1.reshape(1, h), beta1.reshape(1, h),
        Win, bin.reshape(1, -1), Wout, bout.reshape(1, h),
    )


# speedup vs baseline: 1.8049x; 1.1081x over previous
import functools

import jax
import jax.numpy as jnp
from jax import lax
from jax.experimental import pallas as pl
from jax.experimental.pallas import tpu as pltpu
from jax.experimental.pallas import tpu_sc as plsc

EPS = 1e-6
SCALE = 30.0
NEG_SLOPE = 0.01

NC = 2
NS = 16
NW = NC * NS
C = 80
NB = 8
L = 16
KS = 4


def _lrelu(x):
    return jnp.where(x >= 0, x, NEG_SLOPE * x)


def _layernorm(x, g, b, h):
    mu = jnp.mean(x, axis=1, keepdims=True)
    d = x - mu
    var = jnp.sum(d * d, axis=1, keepdims=True) / (h - 1)
    sigma = jnp.sqrt(var + EPS)
    return g * d / (sigma + EPS) + b


KG = 5


def _gather(h_v, src, e_total):
    n, h = h_v.shape
    per_w = e_total // NW
    chunks = per_w // C
    iters = chunks // KG
    mesh = plsc.VectorSubcoreMesh(core_axis_name="c", subcore_axis_name="s")

    @functools.partial(
        pl.kernel,
        out_type=jax.ShapeDtypeStruct((e_total, h), jnp.float32),
        mesh=mesh,
        scratch_types=(
            [pltpu.VMEM((C,), jnp.int32) for _ in range(KG)]
            + [pltpu.VMEM((C, h), jnp.float32) for _ in range(KG)]
            + [pltpu.SemaphoreType.DMA]
            + [pltpu.SemaphoreType.DMA for _ in range(2 * KG)]
        ),
    )
    def k(hv_hbm, src_hbm, out_hbm, *scratch):
        idx_v = scratch[:KG]
        rows_v = scratch[KG:2 * KG]
        sem_i = scratch[2 * KG]
        sem_g = scratch[2 * KG + 1:2 * KG + 1 + KG]
        sem_w = scratch[2 * KG + 1 + KG:]
        wid = lax.axis_index("s") * NC + lax.axis_index("c")
        base = wid * per_w

        def body(t, carry):
            e0 = base + t * (KG * C)
            di = []
            for kk in range(KG):
                di.append(pltpu.async_copy(
                    src_hbm.at[pl.ds(e0 + kk * C, C)], idx_v[kk], sem_i))
            for kk in range(KG):
                di[kk].wait()
            dg = []
            for kk in range(KG):
                dg.append(pltpu.async_copy(
                    hv_hbm.at[idx_v[kk]], rows_v[kk], sem_g[kk]))
            dw = []
            for kk in range(KG):
                dg[kk].wait()
                dw.append(pltpu.async_copy(
                    rows_v[kk], out_hbm.at[pl.ds(e0 + kk * C, C)], sem_w[kk]))
            for kk in range(KG):
                dw[kk].wait()
            return carry

        lax.fori_loop(0, iters, body, 0)

    return k(h_v, src)


def _edge_body(hvg_ref, he_ref, w1a_ref, w1b_ref, b1_ref, w2_ref, b2_ref,
               w3_ref, b3_ref, aa_ref, ab_ref, out_ref, att_ref):
    x = hvg_ref[...]
    e = he_ref[...]
    pre = (jnp.dot(x, w1a_ref[...], preferred_element_type=jnp.float32)
           + jnp.dot(e, w1b_ref[...], preferred_element_type=jnp.float32)
           + b1_ref[...])
    h1 = _lrelu(pre)
    h2 = _lrelu(jnp.dot(h1, w2_ref[...], preferred_element_type=jnp.float32)
                + b2_ref[...])
    hm = jnp.dot(h2, w3_ref[...], preferred_element_type=jnp.float32) + b3_ref[...]
    logit = (jnp.sum(x * aa_ref[...], axis=1, keepdims=True)
             + jnp.sum(e * ab_ref[...], axis=1, keepdims=True))
    att = jnp.exp(jax.nn.sigmoid(_lrelu(logit)))
    out_ref[...] = hm * att
    att_ref[...] = att


def _edge_mlp(hvg, h_e, w1a, w1b, b1, w2, b2, w3, b3, aa, ab):
    e_total, h = hvg.shape
    nin = h_e.shape[1]
    be = 8000
    grid = e_total // be
    return pl.pallas_call(
        _edge_body,
        grid=(grid,),
        in_specs=[
            pl.BlockSpec((be, h), lambda i: (i, 0)),
            pl.BlockSpec((be, nin), lambda i: (i, 0)),
            pl.BlockSpec((h, h), lambda i: (0, 0)),
            pl.BlockSpec((nin, h), lambda i: (0, 0)),
            pl.BlockSpec((1, h), lambda i: (0, 0)),
            pl.BlockSpec((h, h), lambda i: (0, 0)),
            pl.BlockSpec((1, h), lambda i: (0, 0)),
            pl.BlockSpec((h, h), lambda i: (0, 0)),
            pl.BlockSpec((1, h), lambda i: (0, 0)),
            pl.BlockSpec((1, h), lambda i: (0, 0)),
            pl.BlockSpec((1, nin), lambda i: (0, 0)),
        ],
        out_specs=[
            pl.BlockSpec((be, h), lambda i: (i, 0)),
            pl.BlockSpec((be, 1), lambda i: (i, 0)),
        ],
        out_shape=[
            jax.ShapeDtypeStruct((e_total, h), jnp.float32),
            jax.ShapeDtypeStruct((e_total, 1), jnp.float32),
        ],
    )(hvg, h_e, w1a, w1b, b1, w2, b2, w3, b3, aa, ab)


def _scatter(payload, src, zeros, n_pad):
    e_total = src.shape[0]
    h = payload.shape[1]
    per_sc = e_total // NC
    per_tile = per_sc // NS
    chunks = per_tile // C
    iters = chunks // KS
    tail = chunks - iters * KS
    n_per_tile = n_pad // NS
    mesh = plsc.VectorSubcoreMesh(core_axis_name="c", subcore_axis_name="s")

    @functools.partial(
        pl.kernel,
        out_type=[
            jax.ShapeDtypeStruct((n_pad, h), jnp.float32),
            jax.ShapeDtypeStruct((n_pad, h), jnp.float32),
        ],
        mesh=mesh,
        scratch_types=(
            [pltpu.VMEM((C,), jnp.int32) for _ in range(KS)]
            + [pltpu.VMEM((C, h), jnp.float32) for _ in range(KS)]
            + [pltpu.VMEM_SHARED((n_pad, h), jnp.float32)]
            + [pltpu.SemaphoreType.DMA, pltpu.SemaphoreType.DMA]
            + [pltpu.SemaphoreType.DMA for _ in range(KS)]
        ),
    )
    def k(pay_hbm, src_hbm, zero_hbm, out_a, out_b, *scratch):
        idx_v = scratch[:KS]
        pay_v = scratch[KS:2 * KS]
        acc = scratch[2 * KS]
        sem_i = scratch[2 * KS + 1]
        sem_p = scratch[2 * KS + 2]
        sem_a = scratch[2 * KS + 3:]
        cid = lax.axis_index("c")
        sid = lax.axis_index("s")
        sl = pl.ds(sid * n_per_tile, n_per_tile)
        pltpu.sync_copy(zero_hbm.at[sl], acc.at[sl])
        plsc.subcore_barrier()
        base = cid * per_sc + sid * per_tile

        def body(t, carry):
            e0 = base + t * (KS * C)
            d = []
            for kk in range(KS):
                d.append(pltpu.async_copy(
                    src_hbm.at[pl.ds(e0 + kk * C, C)], idx_v[kk], sem_i))
                d.append(pltpu.async_copy(
                    pay_hbm.at[pl.ds(e0 + kk * C, C)], pay_v[kk], sem_p))
            for dd in d:
                dd.wait()
            da = []
            for kk in range(KS):
                da.append(pltpu.async_copy(
                    pay_v[kk], acc.at[idx_v[kk]], sem_a[kk], add=True))
            for dd in da:
                dd.wait()
            return carry

        lax.fori_loop(0, iters, body, 0)
        for j in range(tail):
            e0 = base + (iters * KS + j) * C
            pltpu.sync_copy(src_hbm.at[pl.ds(e0, C)], idx_v[0])
            pltpu.sync_copy(pay_hbm.at[pl.ds(e0, C)], pay_v[0])
            pltpu.sync_copy(pay_v[0], acc.at[idx_v[0]], add=True)
        plsc.subcore_barrier()

        @pl.when(cid == 0)
        def _():
            pltpu.sync_copy(acc.at[sl], out_a.at[sl])

        @pl.when(cid == 1)
        def _():
            pltpu.sync_copy(acc.at[sl], out_b.at[sl])

    return k(payload, src, zeros)


def _att_hist(att, src, n_pad):
    e_total = src.shape[0]
    per_tile = e_total // NW
    chunks = per_tile // C
    iters = chunks // KS
    tail = chunks - iters * KS
    mesh = plsc.VectorSubcoreMesh(core_axis_name="c", subcore_axis_name="s")

    @functools.partial(
        pl.kernel,
        out_type=jax.ShapeDtypeStruct((NW, NB, n_pad), jnp.float32),
        mesh=mesh,
        scratch_types=(
            [pltpu.VMEM((C,), jnp.int32) for _ in range(KS)]
            + [pltpu.VMEM((C,), jnp.float32) for _ in range(KS)]
            + [pltpu.VMEM((NB * n_pad,), jnp.float32)]
            + [pltpu.SemaphoreType.DMA, pltpu.SemaphoreType.DMA]
        ),
        compiler_params=pltpu.CompilerParams(needs_layout_passes=False),
    )
    def k(att_hbm, src_hbm, t_out, *scratch):
        idx_v = scratch[:KS]
        att_v = scratch[KS:2 * KS]
        tacc = scratch[2 * KS]
        sem_i = scratch[2 * KS + 1]
        sem_a = scratch[2 * KS + 2]
        cid = lax.axis_index("c")
        sid = lax.axis_index("s")
        wid = sid * NC + cid

        def zbody(i, carry):
            tacc[pl.ds(i * L, L)] = jnp.zeros((L,), jnp.float32)
            return carry

        lax.fori_loop(0, NB * n_pad // L, zbody, 0)

        base = wid * per_tile
        lane = lax.iota(jnp.int32, L)
        bank_off = (lane % NB) * n_pad
        m_lo = lane < NB
        m_hi = lane >= NB

        def hist(kk):
            for g in range(C // L):
                flat = idx_v[kk][pl.ds(g * L, L)] + bank_off
                att16 = att_v[kk][pl.ds(g * L, L)]
                plsc.addupdate_scatter(tacc, [flat], att16, mask=m_lo)
                plsc.addupdate_scatter(tacc, [flat], att16, mask=m_hi)

        def body(t, carry):
            e0 = base + t * (KS * C)
            d = []
            for kk in range(KS):
                d.append(pltpu.async_copy(
                    src_hbm.at[pl.ds(e0 + kk * C, C)], idx_v[kk], sem_i))
                d.append(pltpu.async_copy(
                    att_hbm.at[pl.ds(e0 + kk * C, C)], att_v[kk], sem_a))
            for dd in d:
                dd.wait()
            for kk in range(KS):
                hist(kk)
            return carry

        lax.fori_loop(0, iters, body, 0)
        for j in range(tail):
            e0 = base + (iters * KS + j) * C
            pltpu.sync_copy(src_hbm.at[pl.ds(e0, C)], idx_v[0])
            pltpu.sync_copy(att_hbm.at[pl.ds(e0, C)], att_v[0])
            hist(0)

        for b in range(NB):
            pltpu.sync_copy(tacc.at[pl.ds(b * n_pad, n_pad)], t_out.at[wid, b])

    return k(att, src)


def _final_body(hv_ref, acca_ref, accb_ref, t_ref,
                g0_ref, beta0_ref, g1_ref, beta1_ref, win_ref, bin_ref,
                wout_ref, bout_ref, out_ref):
    h = hv_ref.shape[1]
    s = acca_ref[...] + accb_ref[...]
    t = jnp.sum(t_ref[...], axis=1, keepdims=True)
    dh = jnp.where(t > 0, s / jnp.where(t > 0, t, 1.0), 0.0) / SCALE
    x = _layernorm(hv_ref[...] + dh, g0_ref[...], beta0_ref[...], h)
    y = jnp.maximum(
        jnp.dot(x, win_ref[...], preferred_element_type=jnp.float32)
        + bin_ref[...], 0.0)
    y = jnp.dot(y, wout_ref[...], preferred_element_type=jnp.float32) + bout_ref[...]
    out_ref[...] = _layernorm(x + y, g1_ref[...], beta1_ref[...], h)


def _finalize(h_v, accs, t_part, g0, beta0, g1, beta1, win, bin_,
              wout, bout):
    n, h = h_v.shape
    bn = 1000
    grid = n // bn
    h4 = win.shape[1]
    kp = t_part.shape[1]
    return pl.pallas_call(
        _final_body,
        grid=(grid,),
        in_specs=[
            pl.BlockSpec((bn, h), lambda i: (i, 0)),
            pl.BlockSpec((bn, h), lambda i: (i, 0)),
            pl.BlockSpec((bn, h), lambda i: (i, 0)),
            pl.BlockSpec((bn, kp), lambda i: (i, 0)),
            pl.BlockSpec((1, h), lambda i: (0, 0)),
            pl.BlockSpec((1, h), lambda i: (0, 0)),
            pl.BlockSpec((1, h), lambda i: (0, 0)),
            pl.BlockSpec((1, h), lambda i: (0, 0)),
            pl.BlockSpec((h, h4), lambda i: (0, 0)),
            pl.BlockSpec((1, h4), lambda i: (0, 0)),
            pl.BlockSpec((h4, h), lambda i: (0, 0)),
            pl.BlockSpec((1, h), lambda i: (0, 0)),
        ],
        out_specs=pl.BlockSpec((bn, h), lambda i: (i, 0)),
        out_shape=jax.ShapeDtypeStruct((n, h), jnp.float32),
    )(h_v, *accs, t_part, g0, beta0, g1, beta1, win, bin_, wout, bout)


def kernel(h_V, h_E, edge_idx, W1, b1, W2, b2, W3, b3, A, g0, beta0, g1,
           beta1, Win, bin, Wout, bout):
    n, h = h_V.shape
    e_total, nin = h_E.shape
    src = edge_idx[0]
    n_pad = ((n + NS * 8 - 1) // (NS * 8)) * NS * 8
    zeros = jnp.zeros((n_pad, h), jnp.float32)
    ws = (W1[:h], W1[h:], b1.reshape(1, h), W2, b2.reshape(1, h),
          W3, b3.reshape(1, h), A[:h].reshape(1, h), A[h:].reshape(1, nin))

    hvg = _gather(h_V, src, e_total)
    weighted, att = _edge_mlp(hvg, h_E, *ws)
    acc_a, acc_b = _scatter(weighted, src, zeros, n_pad)
    t_out = _att_hist(att.reshape(e_total), src, n_pad)
    t_part = t_out.reshape(NW * NB, n_pad).T
    return _finalize(
        h_V, [acc_a[:n], acc_b[:n]], t_part[:n],
        g0.reshape(1, h), beta0.reshape(1, h),
        g1.reshape(1, h), beta1.reshape(1, h),
        Win, bin.reshape(1, -1), Wout, bout.reshape(1, h),
    )


# speedup vs baseline: 1.8269x; 1.0122x over previous
import functools

import jax
import jax.numpy as jnp
from jax import lax
from jax.experimental import pallas as pl
from jax.experimental.pallas import tpu as pltpu
from jax.experimental.pallas import tpu_sc as plsc

EPS = 1e-6
SCALE = 30.0
NEG_SLOPE = 0.01

NC = 2
NS = 16
NW = NC * NS
C = 80
NB = 8
L = 16
KS = 4


def _lrelu(x):
    return jnp.where(x >= 0, x, NEG_SLOPE * x)


def _layernorm(x, g, b, h):
    mu = jnp.mean(x, axis=1, keepdims=True)
    d = x - mu
    var = jnp.sum(d * d, axis=1, keepdims=True) / (h - 1)
    sigma = jnp.sqrt(var + EPS)
    return g * d / (sigma + EPS) + b


KG = 5


def _gather(h_v, src, e_total):
    n, h = h_v.shape
    per_w = e_total // NW
    chunks = per_w // C
    iters = chunks // KG
    mesh = plsc.VectorSubcoreMesh(core_axis_name="c", subcore_axis_name="s")

    @functools.partial(
        pl.kernel,
        out_type=jax.ShapeDtypeStruct((e_total, h), jnp.float32),
        mesh=mesh,
        scratch_types=(
            [pltpu.VMEM((C,), jnp.int32) for _ in range(KG)]
            + [pltpu.VMEM((C, h), jnp.float32) for _ in range(KG)]
            + [pltpu.SemaphoreType.DMA]
            + [pltpu.SemaphoreType.DMA for _ in range(2 * KG)]
        ),
    )
    def k(hv_hbm, src_hbm, out_hbm, *scratch):
        idx_v = scratch[:KG]
        rows_v = scratch[KG:2 * KG]
        sem_i = scratch[2 * KG]
        sem_g = scratch[2 * KG + 1:2 * KG + 1 + KG]
        sem_w = scratch[2 * KG + 1 + KG:]
        wid = lax.axis_index("s") * NC + lax.axis_index("c")
        base = wid * per_w

        def body(t, carry):
            e0 = base + t * (KG * C)
            di = []
            for kk in range(KG):
                di.append(pltpu.async_copy(
                    src_hbm.at[pl.ds(e0 + kk * C, C)], idx_v[kk], sem_i))
            for kk in range(KG):
                di[kk].wait()
            dg = []
            for kk in range(KG):
                dg.append(pltpu.async_copy(
                    hv_hbm.at[idx_v[kk]], rows_v[kk], sem_g[kk]))
            dw = []
            for kk in range(KG):
                dg[kk].wait()
                dw.append(pltpu.async_copy(
                    rows_v[kk], out_hbm.at[pl.ds(e0 + kk * C, C)], sem_w[kk]))
            for kk in range(KG):
                dw[kk].wait()
            return carry

        lax.fori_loop(0, iters, body, 0)

    return k(h_v, src)


def _edge_body(hvg_ref, he_ref, w1a_ref, w1b_ref, b1_ref, w2_ref, b2_ref,
               w3_ref, b3_ref, aa_ref, ab_ref, out_ref, att_ref):
    x = hvg_ref[...]
    e = he_ref[...]
    bf = jnp.bfloat16
    pre = (jnp.dot(x.astype(bf), w1a_ref[...].astype(bf),
                   preferred_element_type=jnp.float32)
           + jnp.dot(e, w1b_ref[...], preferred_element_type=jnp.float32)
           + b1_ref[...])
    h1 = _lrelu(pre)
    h2 = _lrelu(jnp.dot(h1.astype(bf), w2_ref[...].astype(bf),
                        preferred_element_type=jnp.float32)
                + b2_ref[...])
    hm = (jnp.dot(h2.astype(bf), w3_ref[...].astype(bf),
                  preferred_element_type=jnp.float32) + b3_ref[...])
    logit = (jnp.sum(x * aa_ref[...], axis=1, keepdims=True)
             + jnp.sum(e * ab_ref[...], axis=1, keepdims=True))
    att = jnp.exp(jax.nn.sigmoid(_lrelu(logit)))
    out_ref[...] = hm * att
    att_ref[...] = att


def _edge_mlp(hvg, h_e, w1a, w1b, b1, w2, b2, w3, b3, aa, ab):
    e_total, h = hvg.shape
    nin = h_e.shape[1]
    be = 8000
    grid = e_total // be
    return pl.pallas_call(
        _edge_body,
        grid=(grid,),
        in_specs=[
            pl.BlockSpec((be, h), lambda i: (i, 0)),
            pl.BlockSpec((be, nin), lambda i: (i, 0)),
            pl.BlockSpec((h, h), lambda i: (0, 0)),
            pl.BlockSpec((nin, h), lambda i: (0, 0)),
            pl.BlockSpec((1, h), lambda i: (0, 0)),
            pl.BlockSpec((h, h), lambda i: (0, 0)),
            pl.BlockSpec((1, h), lambda i: (0, 0)),
            pl.BlockSpec((h, h), lambda i: (0, 0)),
            pl.BlockSpec((1, h), lambda i: (0, 0)),
            pl.BlockSpec((1, h), lambda i: (0, 0)),
            pl.BlockSpec((1, nin), lambda i: (0, 0)),
        ],
        out_specs=[
            pl.BlockSpec((be, h), lambda i: (i, 0)),
            pl.BlockSpec((be, 1), lambda i: (i, 0)),
        ],
        out_shape=[
            jax.ShapeDtypeStruct((e_total, h), jnp.float32),
            jax.ShapeDtypeStruct((e_total, 1), jnp.float32),
        ],
    )(hvg, h_e, w1a, w1b, b1, w2, b2, w3, b3, aa, ab)


def _scatter(payload, src, zeros, n_pad):
    e_total = src.shape[0]
    h = payload.shape[1]
    per_sc = e_total // NC
    per_tile = per_sc // NS
    chunks = per_tile // C
    iters = chunks // KS
    tail = chunks - iters * KS
    n_per_tile = n_pad // NS
    mesh = plsc.VectorSubcoreMesh(core_axis_name="c", subcore_axis_name="s")

    @functools.partial(
        pl.kernel,
        out_type=[
            jax.ShapeDtypeStruct((n_pad, h), jnp.float32),
            jax.ShapeDtypeStruct((n_pad, h), jnp.float32),
        ],
        mesh=mesh,
        scratch_types=(
            [pltpu.VMEM((C,), jnp.int32) for _ in range(KS)]
            + [pltpu.VMEM((C, h), jnp.float32) for _ in range(KS)]
            + [pltpu.VMEM_SHARED((n_pad, h), jnp.float32)]
            + [pltpu.SemaphoreType.DMA, pltpu.SemaphoreType.DMA]
            + [pltpu.SemaphoreType.DMA for _ in range(KS)]
        ),
    )
    def k(pay_hbm, src_hbm, zero_hbm, out_a, out_b, *scratch):
        idx_v = scratch[:KS]
        pay_v = scratch[KS:2 * KS]
        acc = scratch[2 * KS]
        sem_i = scratch[2 * KS + 1]
        sem_p = scratch[2 * KS + 2]
        sem_a = scratch[2 * KS + 3:]
        cid = lax.axis_index("c")
        sid = lax.axis_index("s")
        sl = pl.ds(sid * n_per_tile, n_per_tile)
        pltpu.sync_copy(zero_hbm.at[sl], acc.at[sl])
        plsc.subcore_barrier()
        base = cid * per_sc + sid * per_tile

        def body(t, carry):
            e0 = base + t * (KS * C)
            d = []
            for kk in range(KS):
                d.append(pltpu.async_copy(
                    src_hbm.at[pl.ds(e0 + kk * C, C)], idx_v[kk], sem_i))
                d.append(pltpu.async_copy(
                    pay_hbm.at[pl.ds(e0 + kk * C, C)], pay_v[kk], sem_p))
            for dd in d:
                dd.wait()
            da = []
            for kk in range(KS):
                da.append(pltpu.async_copy(
                    pay_v[kk], acc.at[idx_v[kk]], sem_a[kk], add=True))
            for dd in da:
                dd.wait()
            return carry

        lax.fori_loop(0, iters, body, 0)
        for j in range(tail):
            e0 = base + (iters * KS + j) * C
            pltpu.sync_copy(src_hbm.at[pl.ds(e0, C)], idx_v[0])
            pltpu.sync_copy(pay_hbm.at[pl.ds(e0, C)], pay_v[0])
            pltpu.sync_copy(pay_v[0], acc.at[idx_v[0]], add=True)
        plsc.subcore_barrier()

        @pl.when(cid == 0)
        def _():
            pltpu.sync_copy(acc.at[sl], out_a.at[sl])

        @pl.when(cid == 1)
        def _():
            pltpu.sync_copy(acc.at[sl], out_b.at[sl])

    return k(payload, src, zeros)


def _att_hist(att, src, n_pad):
    e_total = src.shape[0]
    per_tile = e_total // NW
    chunks = per_tile // C
    iters = chunks // KS
    tail = chunks - iters * KS
    mesh = plsc.VectorSubcoreMesh(core_axis_name="c", subcore_axis_name="s")

    @functools.partial(
        pl.kernel,
        out_type=jax.ShapeDtypeStruct((NW, NB, n_pad), jnp.float32),
        mesh=mesh,
        scratch_types=(
            [pltpu.VMEM((C,), jnp.int32) for _ in range(KS)]
            + [pltpu.VMEM((C,), jnp.float32) for _ in range(KS)]
            + [pltpu.VMEM((NB * n_pad,), jnp.float32)]
            + [pltpu.SemaphoreType.DMA, pltpu.SemaphoreType.DMA]
        ),
        compiler_params=pltpu.CompilerParams(needs_layout_passes=False),
    )
    def k(att_hbm, src_hbm, t_out, *scratch):
        idx_v = scratch[:KS]
        att_v = scratch[KS:2 * KS]
        tacc = scratch[2 * KS]
        sem_i = scratch[2 * KS + 1]
        sem_a = scratch[2 * KS + 2]
        cid = lax.axis_index("c")
        sid = lax.axis_index("s")
        wid = sid * NC + cid

        def zbody(i, carry):
            tacc[pl.ds(i * L, L)] = jnp.zeros((L,), jnp.float32)
            return carry

        lax.fori_loop(0, NB * n_pad // L, zbody, 0)

        base = wid * per_tile
        lane = lax.iota(jnp.int32, L)
        bank_off = (lane % NB) * n_pad
        m_lo = lane < NB
        m_hi = lane >= NB

        def hist(kk):
            for g in range(C // L):
                flat = idx_v[kk][pl.ds(g * L, L)] + bank_off
                att16 = att_v[kk][pl.ds(g * L, L)]
                plsc.addupdate_scatter(tacc, [flat], att16, mask=m_lo)
                plsc.addupdate_scatter(tacc, [flat], att16, mask=m_hi)

        def body(t, carry):
            e0 = base + t * (KS * C)
            d = []
            for kk in range(KS):
                d.append(pltpu.async_copy(
                    src_hbm.at[pl.ds(e0 + kk * C, C)], idx_v[kk], sem_i))
                d.append(pltpu.async_copy(
                    att_hbm.at[pl.ds(e0 + kk * C, C)], att_v[kk], sem_a))
            for dd in d:
                dd.wait()
            for kk in range(KS):
                hist(kk)
            return carry

        lax.fori_loop(0, iters, body, 0)
        for j in range(tail):
            e0 = base + (iters * KS + j) * C
            pltpu.sync_copy(src_hbm.at[pl.ds(e0, C)], idx_v[0])
            pltpu.sync_copy(att_hbm.at[pl.ds(e0, C)], att_v[0])
            hist(0)

        for b in range(NB):
            pltpu.sync_copy(tacc.at[pl.ds(b * n_pad, n_pad)], t_out.at[wid, b])

    return k(att, src)


def _final_body(hv_ref, acca_ref, accb_ref, t_ref,
                g0_ref, beta0_ref, g1_ref, beta1_ref, win_ref, bin_ref,
                wout_ref, bout_ref, out_ref):
    h = hv_ref.shape[1]
    s = acca_ref[...] + accb_ref[...]
    t = jnp.sum(t_ref[...], axis=1, keepdims=True)
    dh = jnp.where(t > 0, s / jnp.where(t > 0, t, 1.0), 0.0) / SCALE
    x = _layernorm(hv_ref[...] + dh, g0_ref[...], beta0_ref[...], h)
    y = jnp.maximum(
        jnp.dot(x, win_ref[...], preferred_element_type=jnp.float32)
        + bin_ref[...], 0.0)
    y = jnp.dot(y, wout_ref[...], preferred_element_type=jnp.float32) + bout_ref[...]
    out_ref[...] = _layernorm(x + y, g1_ref[...], beta1_ref[...], h)


def _finalize(h_v, accs, t_part, g0, beta0, g1, beta1, win, bin_,
              wout, bout):
    n, h = h_v.shape
    bn = 1000
    grid = n // bn
    h4 = win.shape[1]
    kp = t_part.shape[1]
    return pl.pallas_call(
        _final_body,
        grid=(grid,),
        in_specs=[
            pl.BlockSpec((bn, h), lambda i: (i, 0)),
            pl.BlockSpec((bn, h), lambda i: (i, 0)),
            pl.BlockSpec((bn, h), lambda i: (i, 0)),
            pl.BlockSpec((bn, kp), lambda i: (i, 0)),
            pl.BlockSpec((1, h), lambda i: (0, 0)),
            pl.BlockSpec((1, h), lambda i: (0, 0)),
            pl.BlockSpec((1, h), lambda i: (0, 0)),
            pl.BlockSpec((1, h), lambda i: (0, 0)),
            pl.BlockSpec((h, h4), lambda i: (0, 0)),
            pl.BlockSpec((1, h4), lambda i: (0, 0)),
            pl.BlockSpec((h4, h), lambda i: (0, 0)),
            pl.BlockSpec((1, h), lambda i: (0, 0)),
        ],
        out_specs=pl.BlockSpec((bn, h), lambda i: (i, 0)),
        out_shape=jax.ShapeDtypeStruct((n, h), jnp.float32),
    )(h_v, *accs, t_part, g0, beta0, g1, beta1, win, bin_, wout, bout)


def kernel(h_V, h_E, edge_idx, W1, b1, W2, b2, W3, b3, A, g0, beta0, g1,
           beta1, Win, bin, Wout, bout):
    n, h = h_V.shape
    e_total, nin = h_E.shape
    src = edge_idx[0]
    n_pad = ((n + NS * 8 - 1) // (NS * 8)) * NS * 8
    zeros = jnp.zeros((n_pad, h), jnp.float32)
    ws = (W1[:h], W1[h:], b1.reshape(1, h), W2, b2.reshape(1, h),
          W3, b3.reshape(1, h), A[:h].reshape(1, h), A[h:].reshape(1, nin))

    hvg = _gather(h_V, src, e_total)
    weighted, att = _edge_mlp(hvg, h_E, *ws)
    acc_a, acc_b = _scatter(weighted, src, zeros, n_pad)
    t_out = _att_hist(att.reshape(e_total), src, n_pad)
    t_part = t_out.reshape(NW * NB, n_pad).T
    return _finalize(
        h_V, [acc_a[:n], acc_b[:n]], t_part[:n],
        g0.reshape(1, h), beta0.reshape(1, h),
        g1.reshape(1, h), beta1.reshape(1, h),
        Win, bin.reshape(1, -1), Wout, bout.reshape(1, h),
    )


# speedup vs baseline: 1.8787x; 1.0284x over previous
import functools

import jax
import jax.numpy as jnp
from jax import lax
from jax.experimental import pallas as pl
from jax.experimental.pallas import tpu as pltpu
from jax.experimental.pallas import tpu_sc as plsc

EPS = 1e-6
SCALE = 30.0
NEG_SLOPE = 0.01

NC = 2
NS = 16
NW = NC * NS
C = 80
NB = 8
L = 16
KS = 4


def _lrelu(x):
    return jnp.where(x >= 0, x, NEG_SLOPE * x)


def _layernorm(x, g, b, h):
    mu = jnp.mean(x, axis=1, keepdims=True)
    d = x - mu
    var = jnp.sum(d * d, axis=1, keepdims=True) / (h - 1)
    sigma = jnp.sqrt(var + EPS)
    return g * d / (sigma + EPS) + b


KG = 5


def _gather(h_v, src, e_total):
    n, h = h_v.shape
    per_w = e_total // NW
    chunks = per_w // C
    iters = chunks // KG
    mesh = plsc.VectorSubcoreMesh(core_axis_name="c", subcore_axis_name="s")

    @functools.partial(
        pl.kernel,
        out_type=jax.ShapeDtypeStruct((e_total, h), jnp.float32),
        mesh=mesh,
        scratch_types=(
            [pltpu.VMEM((C,), jnp.int32) for _ in range(KG)]
            + [pltpu.VMEM((C, h), jnp.float32) for _ in range(KG)]
            + [pltpu.SemaphoreType.DMA]
            + [pltpu.SemaphoreType.DMA for _ in range(2 * KG)]
        ),
    )
    def k(hv_hbm, src_hbm, out_hbm, *scratch):
        idx_v = scratch[:KG]
        rows_v = scratch[KG:2 * KG]
        sem_i = scratch[2 * KG]
        sem_g = scratch[2 * KG + 1:2 * KG + 1 + KG]
        sem_w = scratch[2 * KG + 1 + KG:]
        wid = lax.axis_index("s") * NC + lax.axis_index("c")
        base = wid * per_w

        def body(t, carry):
            e0 = base + t * (KG * C)
            ep = e0 - KG * C
            di = []
            for kk in range(KG):
                di.append(pltpu.async_copy(
                    src_hbm.at[pl.ds(e0 + kk * C, C)], idx_v[kk], sem_i))
            for kk in range(KG):
                di[kk].wait()
            dg = []
            for kk in range(KG):
                @pl.when(t > 0)
                def _(kk=kk):
                    pltpu.make_async_copy(
                        rows_v[kk], out_hbm.at[pl.ds(ep + kk * C, C)],
                        sem_w[kk]).wait()
                dg.append(pltpu.async_copy(
                    hv_hbm.at[idx_v[kk]], rows_v[kk], sem_g[kk]))
            for kk in range(KG):
                dg[kk].wait()
                pltpu.async_copy(
                    rows_v[kk], out_hbm.at[pl.ds(e0 + kk * C, C)], sem_w[kk])
            return carry

        lax.fori_loop(0, iters, body, 0)
        el = base + (iters - 1) * (KG * C)
        for kk in range(KG):
            pltpu.make_async_copy(
                rows_v[kk], out_hbm.at[pl.ds(el + kk * C, C)], sem_w[kk]).wait()

    return k(h_v, src)


def _edge_body(hvg_ref, he_ref, w1a_ref, w1b_ref, b1_ref, w2_ref, b2_ref,
               w3_ref, b3_ref, aa_ref, ab_ref, out_ref, att_ref):
    x = hvg_ref[...]
    e = he_ref[...]
    bf = jnp.bfloat16
    pre = (jnp.dot(x.astype(bf), w1a_ref[...].astype(bf),
                   preferred_element_type=jnp.float32)
           + jnp.dot(e, w1b_ref[...], preferred_element_type=jnp.float32)
           + b1_ref[...])
    h1 = _lrelu(pre)
    h2 = _lrelu(jnp.dot(h1.astype(bf), w2_ref[...].astype(bf),
                        preferred_element_type=jnp.float32)
                + b2_ref[...])
    hm = (jnp.dot(h2.astype(bf), w3_ref[...].astype(bf),
                  preferred_element_type=jnp.float32) + b3_ref[...])
    logit = (jnp.sum(x * aa_ref[...], axis=1, keepdims=True)
             + jnp.sum(e * ab_ref[...], axis=1, keepdims=True))
    att = jnp.exp(jax.nn.sigmoid(_lrelu(logit)))
    out_ref[...] = hm * att
    att_ref[...] = att


def _edge_mlp(hvg, h_e, w1a, w1b, b1, w2, b2, w3, b3, aa, ab):
    e_total, h = hvg.shape
    nin = h_e.shape[1]
    be = 8000
    grid = e_total // be
    return pl.pallas_call(
        _edge_body,
        grid=(grid,),
        in_specs=[
            pl.BlockSpec((be, h), lambda i: (i, 0)),
            pl.BlockSpec((be, nin), lambda i: (i, 0)),
            pl.BlockSpec((h, h), lambda i: (0, 0)),
            pl.BlockSpec((nin, h), lambda i: (0, 0)),
            pl.BlockSpec((1, h), lambda i: (0, 0)),
            pl.BlockSpec((h, h), lambda i: (0, 0)),
            pl.BlockSpec((1, h), lambda i: (0, 0)),
            pl.BlockSpec((h, h), lambda i: (0, 0)),
            pl.BlockSpec((1, h), lambda i: (0, 0)),
            pl.BlockSpec((1, h), lambda i: (0, 0)),
            pl.BlockSpec((1, nin), lambda i: (0, 0)),
        ],
        out_specs=[
            pl.BlockSpec((be, h), lambda i: (i, 0)),
            pl.BlockSpec((be, 1), lambda i: (i, 0)),
        ],
        out_shape=[
            jax.ShapeDtypeStruct((e_total, h), jnp.float32),
            jax.ShapeDtypeStruct((e_total, 1), jnp.float32),
        ],
    )(hvg, h_e, w1a, w1b, b1, w2, b2, w3, b3, aa, ab)


def _scatter(payload, src, zeros, n_pad):
    e_total = src.shape[0]
    h = payload.shape[1]
    per_sc = e_total // NC
    per_tile = per_sc // NS
    chunks = per_tile // C
    iters = chunks // KS
    tail = chunks - iters * KS
    n_per_tile = n_pad // NS
    mesh = plsc.VectorSubcoreMesh(core_axis_name="c", subcore_axis_name="s")

    @functools.partial(
        pl.kernel,
        out_type=[
            jax.ShapeDtypeStruct((n_pad, h), jnp.float32),
            jax.ShapeDtypeStruct((n_pad, h), jnp.float32),
        ],
        mesh=mesh,
        scratch_types=(
            [pltpu.VMEM((C,), jnp.int32) for _ in range(KS)]
            + [pltpu.VMEM((C, h), jnp.float32) for _ in range(KS)]
            + [pltpu.VMEM_SHARED((n_pad, h), jnp.float32)]
            + [pltpu.SemaphoreType.DMA, pltpu.SemaphoreType.DMA]
            + [pltpu.SemaphoreType.DMA for _ in range(KS)]
        ),
    )
    def k(pay_hbm, src_hbm, zero_hbm, out_a, out_b, *scratch):
        idx_v = scratch[:KS]
        pay_v = scratch[KS:2 * KS]
        acc = scratch[2 * KS]
        sem_i = scratch[2 * KS + 1]
        sem_p = scratch[2 * KS + 2]
        sem_a = scratch[2 * KS + 3:]
        cid = lax.axis_index("c")
        sid = lax.axis_index("s")
        sl = pl.ds(sid * n_per_tile, n_per_tile)
        pltpu.sync_copy(zero_hbm.at[sl], acc.at[sl])
        plsc.subcore_barrier()
        base = cid * per_sc + sid * per_tile

        def body(t, carry):
            e0 = base + t * (KS * C)
            d = []
            for kk in range(KS):
                @pl.when(t > 0)
                def _(kk=kk):
                    pltpu.make_async_copy(
                        pay_v[kk], acc.at[idx_v[kk]], sem_a[kk]).wait()
                d.append(pltpu.async_copy(
                    src_hbm.at[pl.ds(e0 + kk * C, C)], idx_v[kk], sem_i))
                d.append(pltpu.async_copy(
                    pay_hbm.at[pl.ds(e0 + kk * C, C)], pay_v[kk], sem_p))
            for dd in d:
                dd.wait()
            for kk in range(KS):
                pltpu.async_copy(
                    pay_v[kk], acc.at[idx_v[kk]], sem_a[kk], add=True)
            return carry

        lax.fori_loop(0, iters, body, 0)
        for kk in range(KS):
            pltpu.make_async_copy(
                pay_v[kk], acc.at[idx_v[kk]], sem_a[kk]).wait()
        for j in range(tail):
            e0 = base + (iters * KS + j) * C
            pltpu.sync_copy(src_hbm.at[pl.ds(e0, C)], idx_v[0])
            pltpu.sync_copy(pay_hbm.at[pl.ds(e0, C)], pay_v[0])
            pltpu.sync_copy(pay_v[0], acc.at[idx_v[0]], add=True)
        plsc.subcore_barrier()

        @pl.when(cid == 0)
        def _():
            pltpu.sync_copy(acc.at[sl], out_a.at[sl])

        @pl.when(cid == 1)
        def _():
            pltpu.sync_copy(acc.at[sl], out_b.at[sl])

    return k(payload, src, zeros)


def _att_hist(att, src, n_pad):
    e_total = src.shape[0]
    per_tile = e_total // NW
    chunks = per_tile // C
    iters = chunks // KS
    tail = chunks - iters * KS
    mesh = plsc.VectorSubcoreMesh(core_axis_name="c", subcore_axis_name="s")

    @functools.partial(
        pl.kernel,
        out_type=jax.ShapeDtypeStruct((NW, NB, n_pad), jnp.float32),
        mesh=mesh,
        scratch_types=(
            [pltpu.VMEM((C,), jnp.int32) for _ in range(KS)]
            + [pltpu.VMEM((C,), jnp.float32) for _ in range(KS)]
            + [pltpu.VMEM((NB * n_pad,), jnp.float32)]
            + [pltpu.SemaphoreType.DMA, pltpu.SemaphoreType.DMA]
        ),
        compiler_params=pltpu.CompilerParams(needs_layout_passes=False),
    )
    def k(att_hbm, src_hbm, t_out, *scratch):
        idx_v = scratch[:KS]
        att_v = scratch[KS:2 * KS]
        tacc = scratch[2 * KS]
        sem_i = scratch[2 * KS + 1]
        sem_a = scratch[2 * KS + 2]
        cid = lax.axis_index("c")
        sid = lax.axis_index("s")
        wid = sid * NC + cid

        def zbody(i, carry):
            tacc[pl.ds(i * L, L)] = jnp.zeros((L,), jnp.float32)
            return carry

        lax.fori_loop(0, NB * n_pad // L, zbody, 0)

        base = wid * per_tile
        lane = lax.iota(jnp.int32, L)
        bank_off = (lane % NB) * n_pad
        m_lo = lane < NB
        m_hi = lane >= NB

        def hist(kk):
            for g in range(C // L):
                flat = idx_v[kk][pl.ds(g * L, L)] + bank_off
                att16 = att_v[kk][pl.ds(g * L, L)]
                plsc.addupdate_scatter(tacc, [flat], att16, mask=m_lo)
                plsc.addupdate_scatter(tacc, [flat], att16, mask=m_hi)

        def body(t, carry):
            e0 = base + t * (KS * C)
            d = []
            for kk in range(KS):
                d.append(pltpu.async_copy(
                    src_hbm.at[pl.ds(e0 + kk * C, C)], idx_v[kk], sem_i))
                d.append(pltpu.async_copy(
                    att_hbm.at[pl.ds(e0 + kk * C, C)], att_v[kk], sem_a))
            for dd in d:
                dd.wait()
            for kk in range(KS):
                hist(kk)
            return carry

        lax.fori_loop(0, iters, body, 0)
        for j in range(tail):
            e0 = base + (iters * KS + j) * C
            pltpu.sync_copy(src_hbm.at[pl.ds(e0, C)], idx_v[0])
            pltpu.sync_copy(att_hbm.at[pl.ds(e0, C)], att_v[0])
            hist(0)

        for b in range(NB):
            pltpu.sync_copy(tacc.at[pl.ds(b * n_pad, n_pad)], t_out.at[wid, b])

    return k(att, src)


def _final_body(hv_ref, acca_ref, accb_ref, t_ref,
                g0_ref, beta0_ref, g1_ref, beta1_ref, win_ref, bin_ref,
                wout_ref, bout_ref, out_ref):
    h = hv_ref.shape[1]
    s = acca_ref[...] + accb_ref[...]
    t = jnp.sum(t_ref[...], axis=1, keepdims=True)
    dh = jnp.where(t > 0, s / jnp.where(t > 0, t, 1.0), 0.0) / SCALE
    x = _layernorm(hv_ref[...] + dh, g0_ref[...], beta0_ref[...], h)
    y = jnp.maximum(
        jnp.dot(x, win_ref[...], preferred_element_type=jnp.float32)
        + bin_ref[...], 0.0)
    y = jnp.dot(y, wout_ref[...], preferred_element_type=jnp.float32) + bout_ref[...]
    out_ref[...] = _layernorm(x + y, g1_ref[...], beta1_ref[...], h)


def _finalize(h_v, accs, t_part, g0, beta0, g1, beta1, win, bin_,
              wout, bout):
    n, h = h_v.shape
    bn = 1000
    grid = n // bn
    h4 = win.shape[1]
    kp = t_part.shape[1]
    return pl.pallas_call(
        _final_body,
        grid=(grid,),
        in_specs=[
            pl.BlockSpec((bn, h), lambda i: (i, 0)),
            pl.BlockSpec((bn, h), lambda i: (i, 0)),
            pl.BlockSpec((bn, h), lambda i: (i, 0)),
            pl.BlockSpec((bn, kp), lambda i: (i, 0)),
            pl.BlockSpec((1, h), lambda i: (0, 0)),
            pl.BlockSpec((1, h), lambda i: (0, 0)),
            pl.BlockSpec((1, h), lambda i: (0, 0)),
            pl.BlockSpec((1, h), lambda i: (0, 0)),
            pl.BlockSpec((h, h4), lambda i: (0, 0)),
            pl.BlockSpec((1, h4), lambda i: (0, 0)),
            pl.BlockSpec((h4, h), lambda i: (0, 0)),
            pl.BlockSpec((1, h), lambda i: (0, 0)),
        ],
        out_specs=pl.BlockSpec((bn, h), lambda i: (i, 0)),
        out_shape=jax.ShapeDtypeStruct((n, h), jnp.float32),
    )(h_v, *accs, t_part, g0, beta0, g1, beta1, win, bin_, wout, bout)


def kernel(h_V, h_E, edge_idx, W1, b1, W2, b2, W3, b3, A, g0, beta0, g1,
           beta1, Win, bin, Wout, bout):
    n, h = h_V.shape
    e_total, nin = h_E.shape
    src = edge_idx[0]
    n_pad = ((n + NS * 8 - 1) // (NS * 8)) * NS * 8
    zeros = jnp.zeros((n_pad, h), jnp.float32)
    ws = (W1[:h], W1[h:], b1.reshape(1, h), W2, b2.reshape(1, h),
          W3, b3.reshape(1, h), A[:h].reshape(1, h), A[h:].reshape(1, nin))

    hvg = _gather(h_V, src, e_total)
    weighted, att = _edge_mlp(hvg, h_E, *ws)
    acc_a, acc_b = _scatter(weighted, src, zeros, n_pad)
    t_out = _att_hist(att.reshape(e_total), src, n_pad)
    t_part = t_out.reshape(NW * NB, n_pad).T
    return _finalize(
        h_V, [acc_a[:n], acc_b[:n]], t_part[:n],
        g0.reshape(1, h), beta0.reshape(1, h),
        g1.reshape(1, h), beta1.reshape(1, h),
        Win, bin.reshape(1, -1), Wout, bout.reshape(1, h),
    )


# speedup vs baseline: 1.9053x; 1.0141x over previous
import functools

import jax
import jax.numpy as jnp
from jax import lax
from jax.experimental import pallas as pl
from jax.experimental.pallas import tpu as pltpu
from jax.experimental.pallas import tpu_sc as plsc

EPS = 1e-6
SCALE = 30.0
NEG_SLOPE = 0.01

NC = 2
NS = 16
NW = NC * NS
C = 80
NB = 8
L = 16
KS = 4


def _lrelu(x):
    return jnp.where(x >= 0, x, NEG_SLOPE * x)


def _layernorm(x, g, b, h):
    mu = jnp.mean(x, axis=1, keepdims=True)
    d = x - mu
    var = jnp.sum(d * d, axis=1, keepdims=True) / (h - 1)
    sigma = jnp.sqrt(var + EPS)
    return g * d / (sigma + EPS) + b


KG = 5


def _gather(h_v, src, e_total):
    n, h = h_v.shape
    per_w = e_total // NW
    chunks = per_w // C
    iters = chunks // KG
    mesh = plsc.VectorSubcoreMesh(core_axis_name="c", subcore_axis_name="s")

    @functools.partial(
        pl.kernel,
        out_type=jax.ShapeDtypeStruct((e_total, h), jnp.float32),
        mesh=mesh,
        scratch_types=(
            [pltpu.VMEM((C,), jnp.int32) for _ in range(KG)]
            + [pltpu.VMEM((C, h), jnp.float32) for _ in range(KG)]
            + [pltpu.SemaphoreType.DMA]
            + [pltpu.SemaphoreType.DMA for _ in range(2 * KG)]
        ),
    )
    def k(hv_hbm, src_hbm, out_hbm, *scratch):
        idx_v = scratch[:KG]
        rows_v = scratch[KG:2 * KG]
        sem_i = scratch[2 * KG]
        sem_g = scratch[2 * KG + 1:2 * KG + 1 + KG]
        sem_w = scratch[2 * KG + 1 + KG:]
        wid = lax.axis_index("s") * NC + lax.axis_index("c")
        base = wid * per_w

        def body(t, carry):
            e0 = base + t * (KG * C)
            ep = e0 - KG * C
            di = []
            for kk in range(KG):
                di.append(pltpu.async_copy(
                    src_hbm.at[pl.ds(e0 + kk * C, C)], idx_v[kk], sem_i))
            for kk in range(KG):
                di[kk].wait()
            dg = []
            for kk in range(KG):
                @pl.when(t > 0)
                def _(kk=kk):
                    pltpu.make_async_copy(
                        rows_v[kk], out_hbm.at[pl.ds(ep + kk * C, C)],
                        sem_w[kk]).wait()
                dg.append(pltpu.async_copy(
                    hv_hbm.at[idx_v[kk]], rows_v[kk], sem_g[kk]))
            for kk in range(KG):
                dg[kk].wait()
                pltpu.async_copy(
                    rows_v[kk], out_hbm.at[pl.ds(e0 + kk * C, C)], sem_w[kk])
            return carry

        lax.fori_loop(0, iters, body, 0)
        el = base + (iters - 1) * (KG * C)
        for kk in range(KG):
            pltpu.make_async_copy(
                rows_v[kk], out_hbm.at[pl.ds(el + kk * C, C)], sem_w[kk]).wait()

    return k(h_v, src)


def _edge_body(hvg_ref, he_ref, w1a_ref, w1b_ref, b1_ref, w2_ref, b2_ref,
               w3_ref, b3_ref, aa_ref, ab_ref, out_ref, att_ref):
    x = hvg_ref[...]
    e = he_ref[...]
    bf = jnp.bfloat16
    pre = (jnp.dot(x.astype(bf), w1a_ref[...].astype(bf),
                   preferred_element_type=jnp.float32)
           + jnp.dot(e, w1b_ref[...], preferred_element_type=jnp.float32)
           + b1_ref[...])
    h1 = _lrelu(pre)
    h2 = _lrelu(jnp.dot(h1.astype(bf), w2_ref[...].astype(bf),
                        preferred_element_type=jnp.float32)
                + b2_ref[...])
    hm = (jnp.dot(h2.astype(bf), w3_ref[...].astype(bf),
                  preferred_element_type=jnp.float32) + b3_ref[...])
    logit = (jnp.sum(x * aa_ref[...], axis=1, keepdims=True)
             + jnp.sum(e * ab_ref[...], axis=1, keepdims=True))
    att = jnp.exp(jax.nn.sigmoid(_lrelu(logit)))
    out_ref[...] = hm * att
    att_ref[...] = att


def _edge_mlp(hvg, h_e, w1a, w1b, b1, w2, b2, w3, b3, aa, ab):
    e_total, h = hvg.shape
    nin = h_e.shape[1]
    be = 8000
    grid = e_total // be
    return pl.pallas_call(
        _edge_body,
        grid=(grid,),
        in_specs=[
            pl.BlockSpec((be, h), lambda i: (i, 0)),
            pl.BlockSpec((be, nin), lambda i: (i, 0)),
            pl.BlockSpec((h, h), lambda i: (0, 0)),
            pl.BlockSpec((nin, h), lambda i: (0, 0)),
            pl.BlockSpec((1, h), lambda i: (0, 0)),
            pl.BlockSpec((h, h), lambda i: (0, 0)),
            pl.BlockSpec((1, h), lambda i: (0, 0)),
            pl.BlockSpec((h, h), lambda i: (0, 0)),
            pl.BlockSpec((1, h), lambda i: (0, 0)),
            pl.BlockSpec((1, h), lambda i: (0, 0)),
            pl.BlockSpec((1, nin), lambda i: (0, 0)),
        ],
        out_specs=[
            pl.BlockSpec((be, h), lambda i: (i, 0)),
            pl.BlockSpec((be, 1), lambda i: (i, 0)),
        ],
        out_shape=[
            jax.ShapeDtypeStruct((e_total, h), jnp.float32),
            jax.ShapeDtypeStruct((e_total, 1), jnp.float32),
        ],
    )(hvg, h_e, w1a, w1b, b1, w2, b2, w3, b3, aa, ab)


KSM = 2
NBM = 2


def _scatter_hist(payload, att, src, zeros, n_pad):
    e_total = src.shape[0]
    h = payload.shape[1]
    per_sc = e_total // NC
    per_tile = per_sc // NS
    chunks = per_tile // C
    iters = chunks // KSM
    tail = chunks - iters * KSM
    n_per_tile = n_pad // NS
    mesh = plsc.VectorSubcoreMesh(core_axis_name="c", subcore_axis_name="s")

    @functools.partial(
        pl.kernel,
        out_type=[
            jax.ShapeDtypeStruct((n_pad, h), jnp.float32),
            jax.ShapeDtypeStruct((n_pad, h), jnp.float32),
            jax.ShapeDtypeStruct((NW, NBM, n_pad), jnp.float32),
        ],
        mesh=mesh,
        scratch_types=(
            [pltpu.VMEM((C,), jnp.int32) for _ in range(KSM)]
            + [pltpu.VMEM((C, h), jnp.float32) for _ in range(KSM)]
            + [pltpu.VMEM((C,), jnp.float32) for _ in range(KSM)]
            + [pltpu.VMEM((NBM * n_pad,), jnp.float32)]
            + [pltpu.VMEM_SHARED((n_pad, h), jnp.float32)]
            + [pltpu.SemaphoreType.DMA, pltpu.SemaphoreType.DMA,
               pltpu.SemaphoreType.DMA]
            + [pltpu.SemaphoreType.DMA for _ in range(KSM)]
        ),
        compiler_params=pltpu.CompilerParams(needs_layout_passes=False),
    )
    def k(pay_hbm, att_hbm, src_hbm, zero_hbm, out_a, out_b, t_out, *scratch):
        idx_v = scratch[:KSM]
        pay_v = scratch[KSM:2 * KSM]
        att_v = scratch[2 * KSM:3 * KSM]
        tacc = scratch[3 * KSM]
        acc = scratch[3 * KSM + 1]
        sem_i = scratch[3 * KSM + 2]
        sem_p = scratch[3 * KSM + 3]
        sem_t = scratch[3 * KSM + 4]
        sem_a = scratch[3 * KSM + 5:]
        cid = lax.axis_index("c")
        sid = lax.axis_index("s")
        wid = sid * NC + cid
        sl = pl.ds(sid * n_per_tile, n_per_tile)
        pltpu.sync_copy(zero_hbm.at[sl], acc.at[sl])

        def zbody(i, carry):
            tacc[pl.ds(i * L, L)] = jnp.zeros((L,), jnp.float32)
            return carry

        lax.fori_loop(0, NBM * n_pad // L, zbody, 0)
        plsc.subcore_barrier()
        base = cid * per_sc + sid * per_tile
        lane = lax.iota(jnp.int32, L)
        bank_off = (lane % NBM) * n_pad
        masks = [(lane // NBM) == j for j in range(L // NBM)]

        def hist(kk):
            for g in range(C // L):
                flat = idx_v[kk][pl.ds(g * L, L)] + bank_off
                att16 = att_v[kk][pl.ds(g * L, L)]
                for m in masks:
                    plsc.addupdate_scatter(tacc, [flat], att16, mask=m)

        def body(t, carry):
            e0 = base + t * (KSM * C)
            d = []
            for kk in range(KSM):
                @pl.when(t > 0)
                def _(kk=kk):
                    pltpu.make_async_copy(
                        pay_v[kk], acc.at[idx_v[kk]], sem_a[kk]).wait()
                d.append(pltpu.async_copy(
                    src_hbm.at[pl.ds(e0 + kk * C, C)], idx_v[kk], sem_i))
                d.append(pltpu.async_copy(
                    pay_hbm.at[pl.ds(e0 + kk * C, C)], pay_v[kk], sem_p))
                d.append(pltpu.async_copy(
                    att_hbm.at[pl.ds(e0 + kk * C, C)], att_v[kk], sem_t))
            for dd in d:
                dd.wait()
            for kk in range(KSM):
                pltpu.async_copy(
                    pay_v[kk], acc.at[idx_v[kk]], sem_a[kk], add=True)
            for kk in range(KSM):
                hist(kk)
            return carry

        lax.fori_loop(0, iters, body, 0)
        for kk in range(KSM):
            pltpu.make_async_copy(
                pay_v[kk], acc.at[idx_v[kk]], sem_a[kk]).wait()
        for j in range(tail):
            e0 = base + (iters * KSM + j) * C
            pltpu.sync_copy(src_hbm.at[pl.ds(e0, C)], idx_v[0])
            pltpu.sync_copy(pay_hbm.at[pl.ds(e0, C)], pay_v[0])
            pltpu.sync_copy(att_hbm.at[pl.ds(e0, C)], att_v[0])
            pltpu.sync_copy(pay_v[0], acc.at[idx_v[0]], add=True)
            hist(0)
        plsc.subcore_barrier()

        for b in range(NBM):
            pltpu.sync_copy(tacc.at[pl.ds(b * n_pad, n_pad)], t_out.at[wid, b])

        @pl.when(cid == 0)
        def _():
            pltpu.sync_copy(acc.at[sl], out_a.at[sl])

        @pl.when(cid == 1)
        def _():
            pltpu.sync_copy(acc.at[sl], out_b.at[sl])

    return k(payload, att, src, zeros)


def _final_body(hv_ref, acca_ref, accb_ref, t_ref,
                g0_ref, beta0_ref, g1_ref, beta1_ref, win_ref, bin_ref,
                wout_ref, bout_ref, out_ref):
    h = hv_ref.shape[1]
    s = acca_ref[...] + accb_ref[...]
    t = jnp.sum(t_ref[...], axis=1, keepdims=True)
    dh = jnp.where(t > 0, s / jnp.where(t > 0, t, 1.0), 0.0) / SCALE
    x = _layernorm(hv_ref[...] + dh, g0_ref[...], beta0_ref[...], h)
    y = jnp.maximum(
        jnp.dot(x, win_ref[...], preferred_element_type=jnp.float32)
        + bin_ref[...], 0.0)
    y = jnp.dot(y, wout_ref[...], preferred_element_type=jnp.float32) + bout_ref[...]
    out_ref[...] = _layernorm(x + y, g1_ref[...], beta1_ref[...], h)


def _finalize(h_v, accs, t_part, g0, beta0, g1, beta1, win, bin_,
              wout, bout):
    n, h = h_v.shape
    bn = 1000
    grid = n // bn
    h4 = win.shape[1]
    kp = t_part.shape[1]
    return pl.pallas_call(
        _final_body,
        grid=(grid,),
        in_specs=[
            pl.BlockSpec((bn, h), lambda i: (i, 0)),
            pl.BlockSpec((bn, h), lambda i: (i, 0)),
            pl.BlockSpec((bn, h), lambda i: (i, 0)),
            pl.BlockSpec((bn, kp), lambda i: (i, 0)),
            pl.BlockSpec((1, h), lambda i: (0, 0)),
            pl.BlockSpec((1, h), lambda i: (0, 0)),
            pl.BlockSpec((1, h), lambda i: (0, 0)),
            pl.BlockSpec((1, h), lambda i: (0, 0)),
            pl.BlockSpec((h, h4), lambda i: (0, 0)),
            pl.BlockSpec((1, h4), lambda i: (0, 0)),
            pl.BlockSpec((h4, h), lambda i: (0, 0)),
            pl.BlockSpec((1, h), lambda i: (0, 0)),
        ],
        out_specs=pl.BlockSpec((bn, h), lambda i: (i, 0)),
        out_shape=jax.ShapeDtypeStruct((n, h), jnp.float32),
    )(h_v, *accs, t_part, g0, beta0, g1, beta1, win, bin_, wout, bout)


def kernel(h_V, h_E, edge_idx, W1, b1, W2, b2, W3, b3, A, g0, beta0, g1,
           beta1, Win, bin, Wout, bout):
    n, h = h_V.shape
    e_total, nin = h_E.shape
    src = edge_idx[0]
    n_pad = ((n + NS * 8 - 1) // (NS * 8)) * NS * 8
    zeros = jnp.zeros((n_pad, h), jnp.float32)
    ws = (W1[:h], W1[h:], b1.reshape(1, h), W2, b2.reshape(1, h),
          W3, b3.reshape(1, h), A[:h].reshape(1, h), A[h:].reshape(1, nin))

    hvg = _gather(h_V, src, e_total)
    weighted, att = _edge_mlp(hvg, h_E, *ws)
    acc_a, acc_b, t_out = _scatter_hist(weighted, att.reshape(e_total), src,
                                        zeros, n_pad)
    t_part = t_out.reshape(NW * NBM, n_pad).T
    return _finalize(
        h_V, [acc_a[:n], acc_b[:n]], t_part[:n],
        g0.reshape(1, h), beta0.reshape(1, h),
        g1.reshape(1, h), beta1.reshape(1, h),
        Win, bin.reshape(1, -1), Wout, bout.reshape(1, h),
    )
